# baseline (speedup 1.0000x reference)
import functools

import jax
import jax.numpy as jnp
from jax import lax
from jax.experimental import pallas as pl
from jax.experimental.pallas import tpu as pltpu
from jax.experimental.pallas import tpu_sc as plsc

H = 8
HW = 16
C = 128
NCORES = 2
NSUB = 16
NP = 10240
TROWS = NP // NSUB
NB = 400
BE = 80


def _dense_core(h3, as_ref, ad_ref, asd_ref, ads_ref, gm_ref, i):
    asrc = jnp.sum(h3 * as_ref[...][None], axis=-1)
    adst = jnp.sum(h3 * ad_ref[...][None], axis=-1)
    asd_ref[...] = jnp.concatenate([asrc, adst], axis=-1)
    ads_ref[...] = jnp.concatenate([adst, asrc], axis=-1)
    am = jnp.concatenate([
        jnp.max(asrc, axis=0, keepdims=True),
        jnp.max(adst, axis=0, keepdims=True),
    ], axis=-1)
    am = jnp.broadcast_to(am, (8, HW))

    @pl.when(i == 0)
    def _():
        gm_ref[...] = jnp.full((8, HW), -1e30, jnp.float32)

    gm_ref[...] = jnp.maximum(gm_ref[...], am)


def _dense_body(x_ref, w_ref, as_ref, ad_ref, h_ref, asd_ref, ads_ref, gm_ref):
    i = pl.program_id(0)
    h = jnp.dot(x_ref[...], w_ref[...], preferred_element_type=jnp.float32)
    h3 = h.reshape(NB, H, C)
    h_ref[...] = h3.transpose(1, 0, 2)
    _dense_core(h3, as_ref, ad_ref, asd_ref, ads_ref, gm_ref, i)


def _dense_stage(x, w, att_src, att_dst):
    n = x.shape[0]
    k = x.shape[1]
    return pl.pallas_call(
        _dense_body,
        grid=(n // NB,),
        in_specs=[
            pl.BlockSpec((NB, k), lambda i: (i, 0)),
            pl.BlockSpec((k, H * C), lambda i: (0, 0)),
            pl.BlockSpec((H, C), lambda i: (0, 0)),
            pl.BlockSpec((H, C), lambda i: (0, 0)),
        ],
        out_specs=[
            pl.BlockSpec((H, NB, C), lambda i: (0, i, 0)),
            pl.BlockSpec((NB, HW), lambda i: (i, 0)),
            pl.BlockSpec((NB, HW), lambda i: (i, 0)),
            pl.BlockSpec((8, HW), lambda i: (0, 0)),
        ],
        out_shape=[
            jax.ShapeDtypeStruct((H, n, C), jnp.float32),
            jax.ShapeDtypeStruct((n, HW), jnp.float32),
            jax.ShapeDtypeStruct((n, HW), jnp.float32),
            jax.ShapeDtypeStruct((8, HW), jnp.float32),
        ],
    )(x, w, att_src, att_dst)


def _mid_body(o_ref, b_ref, w_ref, as_ref, ad_ref, h_ref, asd_ref, ads_ref, gm_ref):
    i = pl.program_id(0)
    m = jnp.sum(o_ref[...], axis=0) * (1.0 / H)
    z = jnp.maximum(m + b_ref[...], 0.0)
    h = jnp.dot(z, w_ref[...], preferred_element_type=jnp.float32)
    h3 = h.reshape(NB, H, C)
    h_ref[...] = h3.transpose(1, 0, 2)
    _dense_core(h3, as_ref, ad_ref, asd_ref, ads_ref, gm_ref, i)


def _mid_stage(out_byhead, b, w, att_src, att_dst):
    n = 10000
    return pl.pallas_call(
        _mid_body,
        grid=(n // NB,),
        in_specs=[
            pl.BlockSpec((H, NB, C), lambda i: (0, i, 0)),
            pl.BlockSpec((1, C), lambda i: (0, 0)),
            pl.BlockSpec((C, H * C), lambda i: (0, 0)),
            pl.BlockSpec((H, C), lambda i: (0, 0)),
            pl.BlockSpec((H, C), lambda i: (0, 0)),
        ],
        out_specs=[
            pl.BlockSpec((H, NB, C), lambda i: (0, i, 0)),
            pl.BlockSpec((NB, HW), lambda i: (i, 0)),
            pl.BlockSpec((NB, HW), lambda i: (i, 0)),
            pl.BlockSpec((8, HW), lambda i: (0, 0)),
        ],
        out_shape=[
            jax.ShapeDtypeStruct((H, n, C), jnp.float32),
            jax.ShapeDtypeStruct((n, HW), jnp.float32),
            jax.ShapeDtypeStruct((n, HW), jnp.float32),
            jax.ShapeDtypeStruct((8, HW), jnp.float32),
        ],
    )(out_byhead, b.reshape(1, C), w, att_src, att_dst)


def _mean_body(o_ref, b_ref, z_ref):
    z_ref[...] = jnp.sum(o_ref[...], axis=0) * (1.0 / H) + b_ref[...]


def _mean_stage(out_byhead, b):
    return pl.pallas_call(
        _mean_body,
        grid=(10000 // NB,),
        in_specs=[
            pl.BlockSpec((H, NB, C), lambda i: (0, i, 0)),
            pl.BlockSpec((1, C), lambda i: (0, 0)),
        ],
        out_specs=pl.BlockSpec((NB, C), lambda i: (i, 0)),
        out_shape=jax.ShapeDtypeStruct((10000, C), jnp.float32),
    )(out_byhead, b.reshape(1, C))


def _edge_softmax(asd, ads, src, dst, g16, z16):
    e = src.shape[0]
    ep = e // (NCORES * NSUB)
    nbatch = ep // BE
    mesh = plsc.VectorSubcoreMesh(core_axis_name="c", subcore_axis_name="s")

    @functools.partial(
        pl.kernel,
        out_type=[
            jax.ShapeDtypeStruct((e, HW), jnp.float32),
            jax.ShapeDtypeStruct((NCORES, NP, HW), jnp.float32),
        ],
        mesh=mesh,
        compiler_params=pltpu.CompilerParams(use_tc_tiling_on_sc=False),
        scratch_types=[
            [pltpu.VMEM((BE,), jnp.int32)] * 3,
            [pltpu.VMEM((BE,), jnp.int32)] * 3,
            [pltpu.VMEM((BE, HW), jnp.float32)] * 3,
            [pltpu.VMEM((BE, HW), jnp.float32)] * 3,
            [pltpu.VMEM((BE, HW), jnp.float32)] * 3,
            pltpu.VMEM((16,), jnp.float32),
            pltpu.VMEM_SHARED((NP, HW), jnp.float32),
            [pltpu.SemaphoreType.DMA] * 3,
            [pltpu.SemaphoreType.DMA] * 3,
            [pltpu.SemaphoreType.DMA] * 3,
            [pltpu.SemaphoreType.DMA] * 3,
        ],
    )
    def k(asd_h, ads_h, src_h, dst_h, g_h, z16_h, ex_h, dpart_h,
          idx_s, idx_d, as_v, ad_v, ex_v, g_v, den_sh, semi, sema, semw, sems):
        c = lax.axis_index("c")
        s = lax.axis_index("s")
        wid = c * NSUB + s
        pltpu.sync_copy(z16_h, den_sh.at[pl.ds(s * TROWS, TROWS)])
        pltpu.sync_copy(g_h, g_v)
        plsc.subcore_barrier()
        g = g_v[...]

        def ibase(kk):
            return pl.multiple_of(wid * ep + kk * BE, 8)

        def issue_idx(kk, m):
            pltpu.async_copy(src_h.at[pl.ds(ibase(kk), BE)], idx_s[m], semi[m])
            pltpu.async_copy(dst_h.at[pl.ds(ibase(kk), BE)], idx_d[m], semi[m])

        def wait_idx(kk, m):
            pltpu.make_async_copy(src_h.at[pl.ds(ibase(kk), BE)], idx_s[m], semi[m]).wait()
            pltpu.make_async_copy(dst_h.at[pl.ds(ibase(kk), BE)], idx_d[m], semi[m]).wait()

        def phase(kk, p):
            pn = (p + 2) % 3
            p1 = (p + 1) % 3

            @pl.when(kk > 0)
            def _():
                pltpu.make_async_copy(ex_v[pn], ex_h.at[pl.ds(ibase(kk - 1), BE)], semw[pn]).wait()
                pltpu.make_async_copy(ex_v[pn], den_sh.at[idx_d[pn]], sems[pn]).wait()

            @pl.when(kk + 2 < nbatch)
            def _():
                issue_idx(kk + 2, pn)

            @pl.when(kk + 1 < nbatch)
            def _():
                wait_idx(kk + 1, p1)
                pltpu.async_copy(asd_h.at[idx_s[p1]], as_v[p1], sema[p1])
                pltpu.async_copy(ads_h.at[idx_d[p1]], ad_v[p1], sema[p1])

            pltpu.make_async_copy(asd_h.at[idx_s[p]], as_v[p], sema[p]).wait()
            pltpu.make_async_copy(ads_h.at[idx_d[p]], ad_v[p], sema[p]).wait()

            def chunk(j, _):
                a_s = as_v[p][j, :]
                a_d = ad_v[p][j, :]
                tt = a_s + a_d
                alpha = jnp.where(tt >= 0, tt, 0.2 * tt)
                tb = g + a_d
                bound = jnp.where(tb >= 0, tb, 0.2 * tb)
                ex_v[p][j, :] = jnp.exp(alpha - bound)
                return 0

            lax.fori_loop(0, BE, chunk, 0, unroll=2)
            pltpu.async_copy(ex_v[p], ex_h.at[pl.ds(ibase(kk), BE)], semw[p])
            pltpu.async_copy(ex_v[p], den_sh.at[idx_d[p]], sems[p], add=True)

        issue_idx(0, 0)
        issue_idx(1, 1)
        wait_idx(0, 0)
        pltpu.async_copy(asd_h.at[idx_s[0]], as_v[0], sema[0])
        pltpu.async_copy(ads_h.at[idx_d[0]], ad_v[0], sema[0])

        def triple(tt, _):
            phase(3 * tt, 0)
            phase(3 * tt + 1, 1)
            phase(3 * tt + 2, 2)
            return 0

        lax.fori_loop(0, nbatch // 3, triple, 0)
        for kk in range(nbatch - nbatch % 3, nbatch):
            phase(kk, kk % 3)
        pm = (nbatch - 1) % 3
        pltpu.make_async_copy(ex_v[pm], ex_h.at[pl.ds(ibase(nbatch - 1), BE)], semw[pm]).wait()
        pltpu.make_async_copy(ex_v[pm], den_sh.at[idx_d[pm]], sems[pm]).wait()
        plsc.subcore_barrier()
        pltpu.sync_copy(den_sh.at[pl.ds(s * TROWS, TROWS)],
                        dpart_h.at[c].at[pl.ds(s * TROWS, TROWS)])

    return k(asd, ads, src, dst, g16, z16)


def _coef_stage(ex, dst, dpart):
    e = dst.shape[0]
    ep = e // (NCORES * NSUB)
    nbatch = ep // BE
    mesh = plsc.VectorSubcoreMesh(core_axis_name="c", subcore_axis_name="s")

    @functools.partial(
        pl.kernel,
        out_type=jax.ShapeDtypeStruct((e, HW), jnp.float32),
        mesh=mesh,
        compiler_params=pltpu.CompilerParams(use_tc_tiling_on_sc=False),
        scratch_types=[
            [pltpu.VMEM((BE,), jnp.int32)] * 3,
            [pltpu.VMEM((BE, HW), jnp.float32)] * 3,
            [pltpu.VMEM((BE, HW), jnp.float32)] * 3,
            [pltpu.VMEM((BE, HW), jnp.float32)] * 3,
            [pltpu.SemaphoreType.DMA] * 3,
            [pltpu.SemaphoreType.DMA] * 3,
            [pltpu.SemaphoreType.DMA] * 3,
        ],
    )
    def k(ex_h, dst_h, dpart_h, coef_h, idx_d, ex_v, e0, e1, semi, sema, semw):
        c = lax.axis_index("c")
        s = lax.axis_index("s")
        wid = c * NSUB + s

        def ibase(kk):
            return pl.multiple_of(wid * ep + kk * BE, 8)

        def phase(kk, p):
            pn = (p + 2) % 3
            p1 = (p + 1) % 3

            @pl.when(kk > 0)
            def _():
                pltpu.make_async_copy(ex_v[pn], coef_h.at[pl.ds(ibase(kk - 1), BE)], semw[pn]).wait()

            @pl.when(kk + 2 < nbatch)
            def _():
                pltpu.async_copy(dst_h.at[pl.ds(ibase(kk + 2), BE)], idx_d[pn], semi[pn])

            @pl.when(kk + 1 < nbatch)
            def _():
                pltpu.make_async_copy(dst_h.at[pl.ds(ibase(kk + 1), BE)], idx_d[p1], semi[p1]).wait()
                pltpu.async_copy(ex_h.at[pl.ds(ibase(kk + 1), BE)], ex_v[p1], sema[p1])
                pltpu.async_copy(dpart_h.at[0].at[idx_d[p1]], e0[p1], sema[p1])
                pltpu.async_copy(dpart_h.at[1].at[idx_d[p1]], e1[p1], sema[p1])

            pltpu.make_async_copy(ex_h.at[pl.ds(ibase(kk), BE)], ex_v[p], sema[p]).wait()
            pltpu.make_async_copy(dpart_h.at[0].at[idx_d[p]], e0[p], sema[p]).wait()
            pltpu.make_async_copy(dpart_h.at[1].at[idx_d[p]], e1[p], sema[p]).wait()

            def cdiv(j, _):
                ex_v[p][j, :] = ex_v[p][j, :] / (e0[p][j, :] + e1[p][j, :] + 1e-16)
                return 0

            lax.fori_loop(0, BE, cdiv, 0, unroll=2)
            pltpu.async_copy(ex_v[p], coef_h.at[pl.ds(ibase(kk), BE)], semw[p])

        pltpu.async_copy(dst_h.at[pl.ds(ibase(0), BE)], idx_d[0], semi[0])
        pltpu.async_copy(dst_h.at[pl.ds(ibase(1), BE)], idx_d[1], semi[1])
        pltpu.make_async_copy(dst_h.at[pl.ds(ibase(0), BE)], idx_d[0], semi[0]).wait()
        pltpu.async_copy(ex_h.at[pl.ds(ibase(0), BE)], ex_v[0], sema[0])
        pltpu.async_copy(dpart_h.at[0].at[idx_d[0]], e0[0], sema[0])
        pltpu.async_copy(dpart_h.at[1].at[idx_d[0]], e1[0], sema[0])

        def triple(tt, _):
            phase(3 * tt, 0)
            phase(3 * tt + 1, 1)
            phase(3 * tt + 2, 2)
            return 0

        lax.fori_loop(0, nbatch // 3, triple, 0)
        for kk in range(nbatch - nbatch % 3, nbatch):
            phase(kk, kk % 3)
        pm = (nbatch - 1) % 3
        pltpu.make_async_copy(ex_v[pm], coef_h.at[pl.ds(ibase(nbatch - 1), BE)], semw[pm]).wait()

    return k(ex, dst, dpart)


def _edge_message(h_byhead, src, dst, coef, z128):
    e = src.shape[0]
    ep = e // NSUB
    nbatch = ep // BE
    hperc = H // NCORES
    mesh = plsc.VectorSubcoreMesh(core_axis_name="c", subcore_axis_name="s")

    @functools.partial(
        pl.kernel,
        out_type=jax.ShapeDtypeStruct((H, NP, C), jnp.float32),
        mesh=mesh,
        compiler_params=pltpu.CompilerParams(use_tc_tiling_on_sc=False),
        scratch_types=[
            [pltpu.VMEM((BE,), jnp.int32)] * 3,
            [pltpu.VMEM((BE,), jnp.int32)] * 3,
            [pltpu.VMEM((BE, HW), jnp.float32)] * 3,
            [pltpu.VMEM((BE, C), jnp.float32)] * 3,
            pltpu.VMEM_SHARED((NP, C), jnp.float32),
            [pltpu.SemaphoreType.DMA] * 3,
            [pltpu.SemaphoreType.DMA] * 3,
            [pltpu.SemaphoreType.DMA] * 3,
        ],
    )
    def k(h_h, src_h, dst_h, coef_h, z128_h, out_h,
          idx_s, idx_d, cf, hr, acc_sh, semi, semh, sems):
        c = lax.axis_index("c")
        s = lax.axis_index("s")

        def issue_idx(kk, m):
            base = pl.multiple_of(s * ep + kk * BE, 8)
            pltpu.async_copy(src_h.at[pl.ds(base, BE)], idx_s[m], semi[m])
            pltpu.async_copy(dst_h.at[pl.ds(base, BE)], idx_d[m], semi[m])
            pltpu.async_copy(coef_h.at[pl.ds(base, BE)], cf[m], semi[m])

        def wait_idx(kk, m):
            base = pl.multiple_of(s * ep + kk * BE, 8)
            pltpu.make_async_copy(src_h.at[pl.ds(base, BE)], idx_s[m], semi[m]).wait()
            pltpu.make_async_copy(dst_h.at[pl.ds(base, BE)], idx_d[m], semi[m]).wait()
            pltpu.make_async_copy(coef_h.at[pl.ds(base, BE)], cf[m], semi[m]).wait()

        for hi in range(hperc):
            hp = c * hperc + hi
            pltpu.sync_copy(z128_h, acc_sh.at[pl.ds(s * TROWS, TROWS)])
            plsc.subcore_barrier()

            def gissue(hh, m):
                pltpu.async_copy(h_h.at[hp].at[idx_s[m]], hr[m], semh[m])

            def gwait(m):
                pltpu.make_async_copy(h_h.at[hp].at[idx_s[m]], hr[m], semh[m]).wait()

            def sissue(m):
                pltpu.async_copy(hr[m], acc_sh.at[idx_d[m]], sems[m], add=True)

            def swait(m):
                pltpu.make_async_copy(hr[m], acc_sh.at[idx_d[m]], sems[m]).wait()

            def compute(m):
                def edge(ei, _):
                    crow = cf[m][ei, :]
                    cb = crow.at[jnp.broadcast_to(hp, (16,))].get(
                        mode="promise_in_bounds")
                    for r in range(C // 16):
                        hr[m][ei, pl.ds(r * 16, 16)] = hr[m][ei, pl.ds(r * 16, 16)] * cb
                    return 0

                lax.fori_loop(0, BE, edge, 0, unroll=4)

            issue_idx(0, 0)
            issue_idx(1, 1)
            wait_idx(0, 0)
            gissue(hp, 0)

            def phase(kk, p):
                pn = (p + 2) % 3

                @pl.when(kk > 0)
                def _():
                    swait(pn)

                @pl.when(kk + 2 < nbatch)
                def _():
                    issue_idx(kk + 2, pn)

                p1 = (p + 1) % 3

                @pl.when(kk + 1 < nbatch)
                def _():
                    wait_idx(kk + 1, p1)
                    gissue(hp, p1)

                gwait(p)
                compute(p)
                sissue(p)

            def triple(tt, _):
                phase(3 * tt, 0)
                phase(3 * tt + 1, 1)
                phase(3 * tt + 2, 2)
                return 0

            lax.fori_loop(0, nbatch // 3, triple, 0)
            for kk in range(nbatch - nbatch % 3, nbatch):
                phase(kk, kk % 3)
            swait((nbatch - 1) % 3)
            plsc.subcore_barrier()
            pltpu.sync_copy(acc_sh.at[pl.ds(s * TROWS, TROWS)],
                            out_h.at[hp].at[pl.ds(s * TROWS, TROWS)])
            plsc.subcore_barrier()

    return k(h_byhead, src, dst, coef, z128)


def _gat_layer(h_byhead, asd, ads, gm, src, dst, z16, z128):
    ex, dpart = _edge_softmax(asd, ads, src, dst, gm[0], z16)
    coef = _coef_stage(ex, dst, dpart)
    return _edge_message(h_byhead, src, dst, coef, z128)


def kernel(x, edge_index, W1, att_src1, att_dst1, b1, W2, att_src2, att_dst2, b2):
    src = edge_index[0]
    dst = edge_index[1]
    z16 = jnp.zeros((TROWS, HW), jnp.float32)
    z128 = jnp.zeros((TROWS, C), jnp.float32)

    h1, asd1, ads1, gm1 = _dense_stage(x, W1, att_src1, att_dst1)
    out1 = _gat_layer(h1, asd1, ads1, gm1, src, dst, z16, z128)
    h2, asd2, ads2, gm2 = _mid_stage(out1, b1, W2, att_src2, att_dst2)
    out2 = _gat_layer(h2, asd2, ads2, gm2, src, dst, z16, z128)
    return _mean_stage(out2, b2)


# speedup vs baseline: 2.0182x; 2.0182x over previous
import functools

import jax
import jax.numpy as jnp
from jax import lax
from jax.experimental import pallas as pl
from jax.experimental.pallas import tpu as pltpu
from jax.experimental.pallas import tpu_sc as plsc

H = 8
HW = 16
C = 128
NCORES = 2
NSUB = 16
NP = 10240
TROWS = NP // NSUB
NB = 400
BE = 80


def _dense_core(h3, as_ref, ad_ref, asd_ref, ads_ref, gm_ref, i):
    asrc = jnp.sum(h3 * as_ref[...][None], axis=-1)
    adst = jnp.sum(h3 * ad_ref[...][None], axis=-1)
    asd_ref[...] = jnp.concatenate([asrc, adst], axis=-1)
    ads_ref[...] = jnp.concatenate([adst, asrc], axis=-1)
    am = jnp.concatenate([
        jnp.max(asrc, axis=0, keepdims=True),
        jnp.max(adst, axis=0, keepdims=True),
    ], axis=-1)
    am = jnp.broadcast_to(am, (8, HW))

    @pl.when(i == 0)
    def _():
        gm_ref[...] = jnp.full((8, HW), -1e30, jnp.float32)

    gm_ref[...] = jnp.maximum(gm_ref[...], am)


def _dense_body(x_ref, w_ref, as_ref, ad_ref, h_ref, asd_ref, ads_ref, gm_ref):
    i = pl.program_id(0)
    h = jnp.dot(x_ref[...], w_ref[...], preferred_element_type=jnp.float32)
    h3 = h.reshape(NB, H, C)
    h_ref[...] = h3.transpose(1, 0, 2)
    _dense_core(h3, as_ref, ad_ref, asd_ref, ads_ref, gm_ref, i)


def _dense_stage(x, w, att_src, att_dst):
    n = x.shape[0]
    k = x.shape[1]
    return pl.pallas_call(
        _dense_body,
        grid=(n // NB,),
        in_specs=[
            pl.BlockSpec((NB, k), lambda i: (i, 0)),
            pl.BlockSpec((k, H * C), lambda i: (0, 0)),
            pl.BlockSpec((H, C), lambda i: (0, 0)),
            pl.BlockSpec((H, C), lambda i: (0, 0)),
        ],
        out_specs=[
            pl.BlockSpec((H, NB, C), lambda i: (0, i, 0)),
            pl.BlockSpec((NB, HW), lambda i: (i, 0)),
            pl.BlockSpec((NB, HW), lambda i: (i, 0)),
            pl.BlockSpec((8, HW), lambda i: (0, 0)),
        ],
        out_shape=[
            jax.ShapeDtypeStruct((H, n, C), jnp.float32),
            jax.ShapeDtypeStruct((n, HW), jnp.float32),
            jax.ShapeDtypeStruct((n, HW), jnp.float32),
            jax.ShapeDtypeStruct((8, HW), jnp.float32),
        ],
    )(x, w, att_src, att_dst)


def _mid_body(o_ref, b_ref, w_ref, as_ref, ad_ref, h_ref, asd_ref, ads_ref, gm_ref):
    i = pl.program_id(0)
    m = jnp.sum(o_ref[...], axis=0) * (1.0 / H)
    z = jnp.maximum(m + b_ref[...], 0.0)
    h = jnp.dot(z, w_ref[...], preferred_element_type=jnp.float32)
    h3 = h.reshape(NB, H, C)
    h_ref[...] = h3.transpose(1, 0, 2)
    _dense_core(h3, as_ref, ad_ref, asd_ref, ads_ref, gm_ref, i)


def _mid_stage(out_byhead, b, w, att_src, att_dst):
    n = 10000
    return pl.pallas_call(
        _mid_body,
        grid=(n // NB,),
        in_specs=[
            pl.BlockSpec((H, NB, C), lambda i: (0, i, 0)),
            pl.BlockSpec((1, C), lambda i: (0, 0)),
            pl.BlockSpec((C, H * C), lambda i: (0, 0)),
            pl.BlockSpec((H, C), lambda i: (0, 0)),
            pl.BlockSpec((H, C), lambda i: (0, 0)),
        ],
        out_specs=[
            pl.BlockSpec((H, NB, C), lambda i: (0, i, 0)),
            pl.BlockSpec((NB, HW), lambda i: (i, 0)),
            pl.BlockSpec((NB, HW), lambda i: (i, 0)),
            pl.BlockSpec((8, HW), lambda i: (0, 0)),
        ],
        out_shape=[
            jax.ShapeDtypeStruct((H, n, C), jnp.float32),
            jax.ShapeDtypeStruct((n, HW), jnp.float32),
            jax.ShapeDtypeStruct((n, HW), jnp.float32),
            jax.ShapeDtypeStruct((8, HW), jnp.float32),
        ],
    )(out_byhead, b.reshape(1, C), w, att_src, att_dst)


def _mean_body(o_ref, b_ref, z_ref):
    z_ref[...] = jnp.sum(o_ref[...], axis=0) * (1.0 / H) + b_ref[...]


def _mean_stage(out_byhead, b):
    return pl.pallas_call(
        _mean_body,
        grid=(10000 // NB,),
        in_specs=[
            pl.BlockSpec((H, NB, C), lambda i: (0, i, 0)),
            pl.BlockSpec((1, C), lambda i: (0, 0)),
        ],
        out_specs=pl.BlockSpec((NB, C), lambda i: (i, 0)),
        out_shape=jax.ShapeDtypeStruct((10000, C), jnp.float32),
    )(out_byhead, b.reshape(1, C))


def _edge_softmax(asd, ads, src, dst, g16, z16):
    e = src.shape[0]
    ep = e // (NCORES * NSUB)
    nbatch = ep // BE
    mesh = plsc.VectorSubcoreMesh(core_axis_name="c", subcore_axis_name="s")

    @functools.partial(
        pl.kernel,
        out_type=[
            jax.ShapeDtypeStruct((e, HW), jnp.float32),
            jax.ShapeDtypeStruct((NCORES, NP, HW), jnp.float32),
        ],
        mesh=mesh,
        compiler_params=pltpu.CompilerParams(use_tc_tiling_on_sc=False),
        scratch_types=[
            [pltpu.VMEM((BE,), jnp.int32)] * 3,
            [pltpu.VMEM((BE,), jnp.int32)] * 3,
            [pltpu.VMEM((BE, HW), jnp.float32)] * 3,
            [pltpu.VMEM((BE, HW), jnp.float32)] * 3,
            [pltpu.VMEM((BE, HW), jnp.float32)] * 3,
            pltpu.VMEM((16,), jnp.float32),
            pltpu.VMEM_SHARED((NP, HW), jnp.float32),
            [pltpu.SemaphoreType.DMA] * 3,
            [pltpu.SemaphoreType.DMA] * 3,
            [pltpu.SemaphoreType.DMA] * 3,
            [pltpu.SemaphoreType.DMA] * 3,
        ],
    )
    def k(asd_h, ads_h, src_h, dst_h, g_h, z16_h, ex_h, dpart_h,
          idx_s, idx_d, as_v, ad_v, ex_v, g_v, den_sh, semi, sema, semw, sems):
        c = lax.axis_index("c")
        s = lax.axis_index("s")
        wid = c * NSUB + s
        pltpu.sync_copy(z16_h, den_sh.at[pl.ds(s * TROWS, TROWS)])
        pltpu.sync_copy(g_h, g_v)
        plsc.subcore_barrier()
        g = g_v[...]

        def ibase(kk):
            return pl.multiple_of(wid * ep + kk * BE, 8)

        def issue_idx(kk, m):
            pltpu.async_copy(src_h.at[pl.ds(ibase(kk), BE)], idx_s[m], semi[m])
            pltpu.async_copy(dst_h.at[pl.ds(ibase(kk), BE)], idx_d[m], semi[m])

        def wait_idx(kk, m):
            pltpu.make_async_copy(src_h.at[pl.ds(ibase(kk), BE)], idx_s[m], semi[m]).wait()
            pltpu.make_async_copy(dst_h.at[pl.ds(ibase(kk), BE)], idx_d[m], semi[m]).wait()

        def phase(kk, p):
            pn = (p + 2) % 3
            p1 = (p + 1) % 3

            @pl.when(kk > 0)
            def _():
                pltpu.make_async_copy(ex_v[pn], ex_h.at[pl.ds(ibase(kk - 1), BE)], semw[pn]).wait()
                pltpu.make_async_copy(ex_v[pn], den_sh.at[idx_d[pn]], sems[pn]).wait()

            @pl.when(kk + 2 < nbatch)
            def _():
                issue_idx(kk + 2, pn)

            @pl.when(kk + 1 < nbatch)
            def _():
                wait_idx(kk + 1, p1)
                pltpu.async_copy(asd_h.at[idx_s[p1]], as_v[p1], sema[p1])
                pltpu.async_copy(ads_h.at[idx_d[p1]], ad_v[p1], sema[p1])

            pltpu.make_async_copy(asd_h.at[idx_s[p]], as_v[p], sema[p]).wait()
            pltpu.make_async_copy(ads_h.at[idx_d[p]], ad_v[p], sema[p]).wait()

            def chunk(j, _):
                a_s = as_v[p][j, :]
                a_d = ad_v[p][j, :]
                tt = a_s + a_d
                alpha = jnp.where(tt >= 0, tt, 0.2 * tt)
                tb = g + a_d
                bound = jnp.where(tb >= 0, tb, 0.2 * tb)
                ex_v[p][j, :] = jnp.exp(alpha - bound)
                return 0

            lax.fori_loop(0, BE, chunk, 0, unroll=2)
            pltpu.async_copy(ex_v[p], ex_h.at[pl.ds(ibase(kk), BE)], semw[p])
            pltpu.async_copy(ex_v[p], den_sh.at[idx_d[p]], sems[p], add=True)

        issue_idx(0, 0)
        issue_idx(1, 1)
        wait_idx(0, 0)
        pltpu.async_copy(asd_h.at[idx_s[0]], as_v[0], sema[0])
        pltpu.async_copy(ads_h.at[idx_d[0]], ad_v[0], sema[0])

        def triple(tt, _):
            phase(3 * tt, 0)
            phase(3 * tt + 1, 1)
            phase(3 * tt + 2, 2)
            return 0

        lax.fori_loop(0, nbatch // 3, triple, 0)
        for kk in range(nbatch - nbatch % 3, nbatch):
            phase(kk, kk % 3)
        pm = (nbatch - 1) % 3
        pltpu.make_async_copy(ex_v[pm], ex_h.at[pl.ds(ibase(nbatch - 1), BE)], semw[pm]).wait()
        pltpu.make_async_copy(ex_v[pm], den_sh.at[idx_d[pm]], sems[pm]).wait()
        plsc.subcore_barrier()
        pltpu.sync_copy(den_sh.at[pl.ds(s * TROWS, TROWS)],
                        dpart_h.at[c].at[pl.ds(s * TROWS, TROWS)])

    return k(asd, ads, src, dst, g16, z16)


def _coef_stage(ex, dst, dpart):
    e = dst.shape[0]
    ep = e // (NCORES * NSUB)
    nbatch = ep // BE
    mesh = plsc.VectorSubcoreMesh(core_axis_name="c", subcore_axis_name="s")

    @functools.partial(
        pl.kernel,
        out_type=jax.ShapeDtypeStruct((e, HW), jnp.float32),
        mesh=mesh,
        compiler_params=pltpu.CompilerParams(use_tc_tiling_on_sc=False),
        scratch_types=[
            [pltpu.VMEM((BE,), jnp.int32)] * 3,
            [pltpu.VMEM((BE, HW), jnp.float32)] * 3,
            [pltpu.VMEM((BE, HW), jnp.float32)] * 3,
            [pltpu.VMEM((BE, HW), jnp.float32)] * 3,
            [pltpu.SemaphoreType.DMA] * 3,
            [pltpu.SemaphoreType.DMA] * 3,
            [pltpu.SemaphoreType.DMA] * 3,
        ],
    )
    def k(ex_h, dst_h, dpart_h, coef_h, idx_d, ex_v, e0, e1, semi, sema, semw):
        c = lax.axis_index("c")
        s = lax.axis_index("s")
        wid = c * NSUB + s

        def ibase(kk):
            return pl.multiple_of(wid * ep + kk * BE, 8)

        def phase(kk, p):
            pn = (p + 2) % 3
            p1 = (p + 1) % 3

            @pl.when(kk > 0)
            def _():
                pltpu.make_async_copy(ex_v[pn], coef_h.at[pl.ds(ibase(kk - 1), BE)], semw[pn]).wait()

            @pl.when(kk + 2 < nbatch)
            def _():
                pltpu.async_copy(dst_h.at[pl.ds(ibase(kk + 2), BE)], idx_d[pn], semi[pn])

            @pl.when(kk + 1 < nbatch)
            def _():
                pltpu.make_async_copy(dst_h.at[pl.ds(ibase(kk + 1), BE)], idx_d[p1], semi[p1]).wait()
                pltpu.async_copy(ex_h.at[pl.ds(ibase(kk + 1), BE)], ex_v[p1], sema[p1])
                pltpu.async_copy(dpart_h.at[0].at[idx_d[p1]], e0[p1], sema[p1])
                pltpu.async_copy(dpart_h.at[1].at[idx_d[p1]], e1[p1], sema[p1])

            pltpu.make_async_copy(ex_h.at[pl.ds(ibase(kk), BE)], ex_v[p], sema[p]).wait()
            pltpu.make_async_copy(dpart_h.at[0].at[idx_d[p]], e0[p], sema[p]).wait()
            pltpu.make_async_copy(dpart_h.at[1].at[idx_d[p]], e1[p], sema[p]).wait()

            def cdiv(j, _):
                ex_v[p][j, :] = ex_v[p][j, :] / (e0[p][j, :] + e1[p][j, :] + 1e-16)
                return 0

            lax.fori_loop(0, BE, cdiv, 0, unroll=2)
            pltpu.async_copy(ex_v[p], coef_h.at[pl.ds(ibase(kk), BE)], semw[p])

        pltpu.async_copy(dst_h.at[pl.ds(ibase(0), BE)], idx_d[0], semi[0])
        pltpu.async_copy(dst_h.at[pl.ds(ibase(1), BE)], idx_d[1], semi[1])
        pltpu.make_async_copy(dst_h.at[pl.ds(ibase(0), BE)], idx_d[0], semi[0]).wait()
        pltpu.async_copy(ex_h.at[pl.ds(ibase(0), BE)], ex_v[0], sema[0])
        pltpu.async_copy(dpart_h.at[0].at[idx_d[0]], e0[0], sema[0])
        pltpu.async_copy(dpart_h.at[1].at[idx_d[0]], e1[0], sema[0])

        def triple(tt, _):
            phase(3 * tt, 0)
            phase(3 * tt + 1, 1)
            phase(3 * tt + 2, 2)
            return 0

        lax.fori_loop(0, nbatch // 3, triple, 0)
        for kk in range(nbatch - nbatch % 3, nbatch):
            phase(kk, kk % 3)
        pm = (nbatch - 1) % 3
        pltpu.make_async_copy(ex_v[pm], coef_h.at[pl.ds(ibase(nbatch - 1), BE)], semw[pm]).wait()

    return k(ex, dst, dpart)


def _edge_message(h_byhead, src, dst, coef, z128):
    e = src.shape[0]
    ep = e // NSUB
    nbatch = ep // BE
    hperc = H // NCORES
    mesh = plsc.VectorSubcoreMesh(core_axis_name="c", subcore_axis_name="s")

    @functools.partial(
        pl.kernel,
        out_type=jax.ShapeDtypeStruct((H, NP, C), jnp.float32),
        mesh=mesh,
        compiler_params=pltpu.CompilerParams(use_tc_tiling_on_sc=False),
        scratch_types=[
            [pltpu.VMEM((BE,), jnp.int32)] * 3,
            [pltpu.VMEM((BE,), jnp.int32)] * 3,
            [pltpu.VMEM((BE, HW), jnp.float32)] * 3,
            [pltpu.VMEM((BE, C), jnp.float32)] * 3,
            pltpu.VMEM_SHARED((NP, C), jnp.float32),
            [pltpu.SemaphoreType.DMA] * 3,
            [pltpu.SemaphoreType.DMA] * 3,
            [pltpu.SemaphoreType.DMA] * 3,
        ],
    )
    def k(h_h, src_h, dst_h, coef_h, z128_h, out_h,
          idx_s, idx_d, cf, hr, acc_sh, semi, semh, sems):
        c = lax.axis_index("c")
        s = lax.axis_index("s")

        def issue_idx(kk, m):
            base = pl.multiple_of(s * ep + kk * BE, 8)
            pltpu.async_copy(src_h.at[pl.ds(base, BE)], idx_s[m], semi[m])
            pltpu.async_copy(dst_h.at[pl.ds(base, BE)], idx_d[m], semi[m])
            pltpu.async_copy(coef_h.at[pl.ds(base, BE)], cf[m], semi[m])

        def wait_idx(kk, m):
            base = pl.multiple_of(s * ep + kk * BE, 8)
            pltpu.make_async_copy(src_h.at[pl.ds(base, BE)], idx_s[m], semi[m]).wait()
            pltpu.make_async_copy(dst_h.at[pl.ds(base, BE)], idx_d[m], semi[m]).wait()
            pltpu.make_async_copy(coef_h.at[pl.ds(base, BE)], cf[m], semi[m]).wait()

        for hi in range(hperc):
            hp = c * hperc + hi
            pltpu.sync_copy(z128_h, acc_sh.at[pl.ds(s * TROWS, TROWS)])
            plsc.subcore_barrier()

            def gissue(hh, m):
                pltpu.async_copy(h_h.at[hp].at[idx_s[m]], hr[m], semh[m])

            def gwait(m):
                pltpu.make_async_copy(h_h.at[hp].at[idx_s[m]], hr[m], semh[m]).wait()

            def sissue(m):
                pltpu.async_copy(hr[m], acc_sh.at[idx_d[m]], sems[m], add=True)

            def swait(m):
                pltpu.make_async_copy(hr[m], acc_sh.at[idx_d[m]], sems[m]).wait()

            def compute(m):
                def edge(ei, _):
                    crow = cf[m][ei, :]
                    cb = crow.at[jnp.broadcast_to(hp, (16,))].get(
                        mode="promise_in_bounds")
                    for r in range(C // 16):
                        hr[m][ei, pl.ds(r * 16, 16)] = hr[m][ei, pl.ds(r * 16, 16)] * cb
                    return 0

                lax.fori_loop(0, BE, edge, 0)

            issue_idx(0, 0)
            issue_idx(1, 1)
            wait_idx(0, 0)
            gissue(hp, 0)

            def phase(kk, p):
                pn = (p + 2) % 3

                @pl.when(kk > 0)
                def _():
                    swait(pn)

                @pl.when(kk + 2 < nbatch)
                def _():
                    issue_idx(kk + 2, pn)

                p1 = (p + 1) % 3

                @pl.when(kk + 1 < nbatch)
                def _():
                    wait_idx(kk + 1, p1)
                    gissue(hp, p1)

                gwait(p)
                compute(p)
                sissue(p)

            def triple(tt, _):
                phase(3 * tt, 0)
                phase(3 * tt + 1, 1)
                phase(3 * tt + 2, 2)
                return 0

            lax.fori_loop(0, nbatch // 3, triple, 0)
            for kk in range(nbatch - nbatch % 3, nbatch):
                phase(kk, kk % 3)
            swait((nbatch - 1) % 3)
            plsc.subcore_barrier()
            pltpu.sync_copy(acc_sh.at[pl.ds(s * TROWS, TROWS)],
                            out_h.at[hp].at[pl.ds(s * TROWS, TROWS)])
            plsc.subcore_barrier()

    return k(h_byhead, src, dst, coef, z128)


def _gat_layer(h_byhead, asd, ads, gm, src, dst, z16, z128):
    ex, dpart = _edge_softmax(asd, ads, src, dst, gm[0], z16)
    coef = _coef_stage(ex, dst, dpart)
    return _edge_message(h_byhead, src, dst, coef, z128)


def kernel(x, edge_index, W1, att_src1, att_dst1, b1, W2, att_src2, att_dst2, b2):
    src = edge_index[0]
    dst = edge_index[1]
    z16 = jnp.zeros((TROWS, HW), jnp.float32)
    z128 = jnp.zeros((TROWS, C), jnp.float32)

    h1, asd1, ads1, gm1 = _dense_stage(x, W1, att_src1, att_dst1)
    out1 = _gat_layer(h1, asd1, ads1, gm1, src, dst, z16, z128)
    h2, asd2, ads2, gm2 = _mid_stage(out1, b1, W2, att_src2, att_dst2)
    out2 = _gat_layer(h2, asd2, ads2, gm2, src, dst, z16, z128)
    return _mean_stage(out2, b2)


# speedup vs baseline: 2.1164x; 1.0487x over previous
import functools

import jax
import jax.numpy as jnp
from jax import lax
from jax.experimental import pallas as pl
from jax.experimental.pallas import tpu as pltpu
from jax.experimental.pallas import tpu_sc as plsc

H = 8
HW = 16
C = 128
NCORES = 2
NSUB = 16
NP = 10240
TROWS = NP // NSUB
NB = 400
BE = 80


def _dense_core(h3, as_ref, ad_ref, asd_ref, ads_ref, gm_ref, i):
    asrc = jnp.sum(h3 * as_ref[...][None], axis=-1)
    adst = jnp.sum(h3 * ad_ref[...][None], axis=-1)
    asd_ref[...] = jnp.concatenate([asrc, adst], axis=-1)
    ads_ref[...] = jnp.concatenate([adst, asrc], axis=-1)
    am = jnp.concatenate([
        jnp.max(asrc, axis=0, keepdims=True),
        jnp.max(adst, axis=0, keepdims=True),
    ], axis=-1)
    am = jnp.broadcast_to(am, (8, HW))

    @pl.when(i == 0)
    def _():
        gm_ref[...] = jnp.full((8, HW), -1e30, jnp.float32)

    gm_ref[...] = jnp.maximum(gm_ref[...], am)


def _dense_body(x_ref, w_ref, as_ref, ad_ref, h_ref, asd_ref, ads_ref, gm_ref):
    i = pl.program_id(0)
    h = jnp.dot(x_ref[...], w_ref[...], preferred_element_type=jnp.float32)
    h3 = h.reshape(NB, H, C)
    h_ref[...] = h3.transpose(1, 0, 2)
    _dense_core(h3, as_ref, ad_ref, asd_ref, ads_ref, gm_ref, i)


def _dense_stage(x, w, att_src, att_dst):
    n = x.shape[0]
    k = x.shape[1]
    return pl.pallas_call(
        _dense_body,
        grid=(n // NB,),
        in_specs=[
            pl.BlockSpec((NB, k), lambda i: (i, 0)),
            pl.BlockSpec((k, H * C), lambda i: (0, 0)),
            pl.BlockSpec((H, C), lambda i: (0, 0)),
            pl.BlockSpec((H, C), lambda i: (0, 0)),
        ],
        out_specs=[
            pl.BlockSpec((H, NB, C), lambda i: (0, i, 0)),
            pl.BlockSpec((NB, HW), lambda i: (i, 0)),
            pl.BlockSpec((NB, HW), lambda i: (i, 0)),
            pl.BlockSpec((8, HW), lambda i: (0, 0)),
        ],
        out_shape=[
            jax.ShapeDtypeStruct((H, n, C), jnp.float32),
            jax.ShapeDtypeStruct((n, HW), jnp.float32),
            jax.ShapeDtypeStruct((n, HW), jnp.float32),
            jax.ShapeDtypeStruct((8, HW), jnp.float32),
        ],
    )(x, w, att_src, att_dst)


def _mid_body(o_ref, b_ref, w_ref, as_ref, ad_ref, h_ref, asd_ref, ads_ref, gm_ref):
    i = pl.program_id(0)
    m = jnp.sum(o_ref[...], axis=0) * (1.0 / H)
    z = jnp.maximum(m + b_ref[...], 0.0)
    h = jnp.dot(z, w_ref[...], preferred_element_type=jnp.float32)
    h3 = h.reshape(NB, H, C)
    h_ref[...] = h3.transpose(1, 0, 2)
    _dense_core(h3, as_ref, ad_ref, asd_ref, ads_ref, gm_ref, i)


def _mid_stage(out_byhead, b, w, att_src, att_dst):
    n = 10000
    return pl.pallas_call(
        _mid_body,
        grid=(n // NB,),
        in_specs=[
            pl.BlockSpec((H, NB, C), lambda i: (0, i, 0)),
            pl.BlockSpec((1, C), lambda i: (0, 0)),
            pl.BlockSpec((C, H * C), lambda i: (0, 0)),
            pl.BlockSpec((H, C), lambda i: (0, 0)),
            pl.BlockSpec((H, C), lambda i: (0, 0)),
        ],
        out_specs=[
            pl.BlockSpec((H, NB, C), lambda i: (0, i, 0)),
            pl.BlockSpec((NB, HW), lambda i: (i, 0)),
            pl.BlockSpec((NB, HW), lambda i: (i, 0)),
            pl.BlockSpec((8, HW), lambda i: (0, 0)),
        ],
        out_shape=[
            jax.ShapeDtypeStruct((H, n, C), jnp.float32),
            jax.ShapeDtypeStruct((n, HW), jnp.float32),
            jax.ShapeDtypeStruct((n, HW), jnp.float32),
            jax.ShapeDtypeStruct((8, HW), jnp.float32),
        ],
    )(out_byhead, b.reshape(1, C), w, att_src, att_dst)


def _mean_body(o_ref, b_ref, z_ref):
    z_ref[...] = jnp.sum(o_ref[...], axis=0) * (1.0 / H) + b_ref[...]


def _mean_stage(out_byhead, b):
    return pl.pallas_call(
        _mean_body,
        grid=(10000 // NB,),
        in_specs=[
            pl.BlockSpec((H, NB, C), lambda i: (0, i, 0)),
            pl.BlockSpec((1, C), lambda i: (0, 0)),
        ],
        out_specs=pl.BlockSpec((NB, C), lambda i: (i, 0)),
        out_shape=jax.ShapeDtypeStruct((10000, C), jnp.float32),
    )(out_byhead, b.reshape(1, C))


def _edge_softmax(asd, ads, src, dst, g16, z16):
    e = src.shape[0]
    ep = e // (NCORES * NSUB)
    nbatch = ep // BE
    mesh = plsc.VectorSubcoreMesh(core_axis_name="c", subcore_axis_name="s")

    @functools.partial(
        pl.kernel,
        out_type=[
            jax.ShapeDtypeStruct((e, HW), jnp.float32),
            jax.ShapeDtypeStruct((NCORES, NP, HW), jnp.float32),
        ],
        mesh=mesh,
        compiler_params=pltpu.CompilerParams(use_tc_tiling_on_sc=False),
        scratch_types=[
            [pltpu.VMEM((BE,), jnp.int32)] * 3,
            [pltpu.VMEM((BE,), jnp.int32)] * 3,
            [pltpu.VMEM((BE, HW), jnp.float32)] * 3,
            [pltpu.VMEM((BE, HW), jnp.float32)] * 3,
            [pltpu.VMEM((BE, HW), jnp.float32)] * 3,
            pltpu.VMEM((16,), jnp.float32),
            pltpu.VMEM_SHARED((NP, HW), jnp.float32),
            [pltpu.SemaphoreType.DMA] * 3,
            [pltpu.SemaphoreType.DMA] * 3,
            [pltpu.SemaphoreType.DMA] * 3,
            [pltpu.SemaphoreType.DMA] * 3,
        ],
    )
    def k(asd_h, ads_h, src_h, dst_h, g_h, z16_h, ex_h, dpart_h,
          idx_s, idx_d, as_v, ad_v, ex_v, g_v, den_sh, semi, sema, semw, sems):
        c = lax.axis_index("c")
        s = lax.axis_index("s")
        wid = c * NSUB + s
        pltpu.sync_copy(z16_h, den_sh.at[pl.ds(s * TROWS, TROWS)])
        pltpu.sync_copy(g_h, g_v)
        plsc.subcore_barrier()
        g = g_v[...]

        def ibase(kk):
            return pl.multiple_of(wid * ep + kk * BE, 8)

        def issue_idx(kk, m):
            pltpu.async_copy(src_h.at[pl.ds(ibase(kk), BE)], idx_s[m], semi[m])
            pltpu.async_copy(dst_h.at[pl.ds(ibase(kk), BE)], idx_d[m], semi[m])

        def wait_idx(kk, m):
            pltpu.make_async_copy(src_h.at[pl.ds(ibase(kk), BE)], idx_s[m], semi[m]).wait()
            pltpu.make_async_copy(dst_h.at[pl.ds(ibase(kk), BE)], idx_d[m], semi[m]).wait()

        def phase(kk, p):
            pn = (p + 2) % 3
            p1 = (p + 1) % 3

            @pl.when(kk > 0)
            def _():
                pltpu.make_async_copy(ex_v[pn], ex_h.at[pl.ds(ibase(kk - 1), BE)], semw[pn]).wait()
                pltpu.make_async_copy(ex_v[pn], den_sh.at[idx_d[pn]], sems[pn]).wait()

            @pl.when(kk + 2 < nbatch)
            def _():
                issue_idx(kk + 2, pn)

            @pl.when(kk + 1 < nbatch)
            def _():
                wait_idx(kk + 1, p1)
                pltpu.async_copy(asd_h.at[idx_s[p1]], as_v[p1], sema[p1])
                pltpu.async_copy(ads_h.at[idx_d[p1]], ad_v[p1], sema[p1])

            pltpu.make_async_copy(asd_h.at[idx_s[p]], as_v[p], sema[p]).wait()
            pltpu.make_async_copy(ads_h.at[idx_d[p]], ad_v[p], sema[p]).wait()

            def chunk(j, _):
                a_s = as_v[p][j, :]
                a_d = ad_v[p][j, :]
                tt = a_s + a_d
                alpha = jnp.where(tt >= 0, tt, 0.2 * tt)
                tb = g + a_d
                bound = jnp.where(tb >= 0, tb, 0.2 * tb)
                ex_v[p][j, :] = jnp.exp(alpha - bound)
                return 0

            lax.fori_loop(0, BE, chunk, 0, unroll=2)
            pltpu.async_copy(ex_v[p], ex_h.at[pl.ds(ibase(kk), BE)], semw[p])
            pltpu.async_copy(ex_v[p], den_sh.at[idx_d[p]], sems[p], add=True)

        issue_idx(0, 0)
        issue_idx(1, 1)
        wait_idx(0, 0)
        pltpu.async_copy(asd_h.at[idx_s[0]], as_v[0], sema[0])
        pltpu.async_copy(ads_h.at[idx_d[0]], ad_v[0], sema[0])

        def triple(tt, _):
            phase(3 * tt, 0)
            phase(3 * tt + 1, 1)
            phase(3 * tt + 2, 2)
            return 0

        lax.fori_loop(0, nbatch // 3, triple, 0)
        for kk in range(nbatch - nbatch % 3, nbatch):
            phase(kk, kk % 3)
        pm = (nbatch - 1) % 3
        pltpu.make_async_copy(ex_v[pm], ex_h.at[pl.ds(ibase(nbatch - 1), BE)], semw[pm]).wait()
        pltpu.make_async_copy(ex_v[pm], den_sh.at[idx_d[pm]], sems[pm]).wait()
        plsc.subcore_barrier()
        pltpu.sync_copy(den_sh.at[pl.ds(s * TROWS, TROWS)],
                        dpart_h.at[c].at[pl.ds(s * TROWS, TROWS)])

    return k(asd, ads, src, dst, g16, z16)


def _coef_stage(ex, dst, dpart):
    e = dst.shape[0]
    ep = e // (NCORES * NSUB)
    nbatch = ep // BE
    mesh = plsc.VectorSubcoreMesh(core_axis_name="c", subcore_axis_name="s")

    @functools.partial(
        pl.kernel,
        out_type=jax.ShapeDtypeStruct((e, HW), jnp.float32),
        mesh=mesh,
        compiler_params=pltpu.CompilerParams(use_tc_tiling_on_sc=False),
        scratch_types=[
            [pltpu.VMEM((BE,), jnp.int32)] * 3,
            [pltpu.VMEM((BE, HW), jnp.float32)] * 3,
            [pltpu.VMEM((BE, HW), jnp.float32)] * 3,
            [pltpu.VMEM((BE, HW), jnp.float32)] * 3,
            [pltpu.SemaphoreType.DMA] * 3,
            [pltpu.SemaphoreType.DMA] * 3,
            [pltpu.SemaphoreType.DMA] * 3,
        ],
    )
    def k(ex_h, dst_h, dpart_h, coef_h, idx_d, ex_v, e0, e1, semi, sema, semw):
        c = lax.axis_index("c")
        s = lax.axis_index("s")
        wid = c * NSUB + s

        def ibase(kk):
            return pl.multiple_of(wid * ep + kk * BE, 8)

        def phase(kk, p):
            pn = (p + 2) % 3
            p1 = (p + 1) % 3

            @pl.when(kk > 0)
            def _():
                pltpu.make_async_copy(ex_v[pn], coef_h.at[pl.ds(ibase(kk - 1), BE)], semw[pn]).wait()

            @pl.when(kk + 2 < nbatch)
            def _():
                pltpu.async_copy(dst_h.at[pl.ds(ibase(kk + 2), BE)], idx_d[pn], semi[pn])

            @pl.when(kk + 1 < nbatch)
            def _():
                pltpu.make_async_copy(dst_h.at[pl.ds(ibase(kk + 1), BE)], idx_d[p1], semi[p1]).wait()
                pltpu.async_copy(ex_h.at[pl.ds(ibase(kk + 1), BE)], ex_v[p1], sema[p1])
                pltpu.async_copy(dpart_h.at[0].at[idx_d[p1]], e0[p1], sema[p1])
                pltpu.async_copy(dpart_h.at[1].at[idx_d[p1]], e1[p1], sema[p1])

            pltpu.make_async_copy(ex_h.at[pl.ds(ibase(kk), BE)], ex_v[p], sema[p]).wait()
            pltpu.make_async_copy(dpart_h.at[0].at[idx_d[p]], e0[p], sema[p]).wait()
            pltpu.make_async_copy(dpart_h.at[1].at[idx_d[p]], e1[p], sema[p]).wait()

            def cdiv(j, _):
                ex_v[p][j, :] = ex_v[p][j, :] / (e0[p][j, :] + e1[p][j, :] + 1e-16)
                return 0

            lax.fori_loop(0, BE, cdiv, 0, unroll=2)
            pltpu.async_copy(ex_v[p], coef_h.at[pl.ds(ibase(kk), BE)], semw[p])

        pltpu.async_copy(dst_h.at[pl.ds(ibase(0), BE)], idx_d[0], semi[0])
        pltpu.async_copy(dst_h.at[pl.ds(ibase(1), BE)], idx_d[1], semi[1])
        pltpu.make_async_copy(dst_h.at[pl.ds(ibase(0), BE)], idx_d[0], semi[0]).wait()
        pltpu.async_copy(ex_h.at[pl.ds(ibase(0), BE)], ex_v[0], sema[0])
        pltpu.async_copy(dpart_h.at[0].at[idx_d[0]], e0[0], sema[0])
        pltpu.async_copy(dpart_h.at[1].at[idx_d[0]], e1[0], sema[0])

        def triple(tt, _):
            phase(3 * tt, 0)
            phase(3 * tt + 1, 1)
            phase(3 * tt + 2, 2)
            return 0

        lax.fori_loop(0, nbatch // 3, triple, 0)
        for kk in range(nbatch - nbatch % 3, nbatch):
            phase(kk, kk % 3)
        pm = (nbatch - 1) % 3
        pltpu.make_async_copy(ex_v[pm], coef_h.at[pl.ds(ibase(nbatch - 1), BE)], semw[pm]).wait()

    return k(ex, dst, dpart)


def _edge_message(h_byhead, src, dst, coef, z128):
    e = src.shape[0]
    ep = e // NSUB
    nbatch = ep // BE
    hperc = H // NCORES
    mesh = plsc.VectorSubcoreMesh(core_axis_name="c", subcore_axis_name="s")

    @functools.partial(
        pl.kernel,
        out_type=jax.ShapeDtypeStruct((H, NP, C), jnp.float32),
        mesh=mesh,
        compiler_params=pltpu.CompilerParams(use_tc_tiling_on_sc=False),
        scratch_types=[
            [pltpu.VMEM((BE,), jnp.int32)] * 3,
            [pltpu.VMEM((BE,), jnp.int32)] * 3,
            [pltpu.VMEM((BE, HW), jnp.float32)] * 3,
            [pltpu.VMEM((BE, C), jnp.float32)] * 3,
            pltpu.VMEM_SHARED((NP, C), jnp.float32),
            [pltpu.SemaphoreType.DMA] * 3,
            [pltpu.SemaphoreType.DMA] * 3,
            [pltpu.SemaphoreType.DMA] * 3,
        ],
    )
    def k(h_h, src_h, dst_h, coef_h, z128_h, out_h,
          idx_s, idx_d, cf, hr, acc_sh, semi, semh, sems):
        c = lax.axis_index("c")
        s = lax.axis_index("s")

        def issue_idx(kk, m):
            base = pl.multiple_of(s * ep + kk * BE, 8)
            pltpu.async_copy(src_h.at[pl.ds(base, BE)], idx_s[m], semi[m])
            pltpu.async_copy(dst_h.at[pl.ds(base, BE)], idx_d[m], semi[m])
            pltpu.async_copy(coef_h.at[pl.ds(base, BE)], cf[m], semi[m])

        def wait_idx(kk, m):
            base = pl.multiple_of(s * ep + kk * BE, 8)
            pltpu.make_async_copy(src_h.at[pl.ds(base, BE)], idx_s[m], semi[m]).wait()
            pltpu.make_async_copy(dst_h.at[pl.ds(base, BE)], idx_d[m], semi[m]).wait()
            pltpu.make_async_copy(coef_h.at[pl.ds(base, BE)], cf[m], semi[m]).wait()

        for hi in range(hperc):
            hp = c * hperc + hi
            pltpu.sync_copy(z128_h, acc_sh.at[pl.ds(s * TROWS, TROWS)])
            plsc.subcore_barrier()

            def gissue(hh, m):
                pltpu.async_copy(h_h.at[hp].at[idx_s[m]], hr[m], semh[m])

            def gwait(m):
                pltpu.make_async_copy(h_h.at[hp].at[idx_s[m]], hr[m], semh[m]).wait()

            def sissue(m):
                pltpu.async_copy(hr[m], acc_sh.at[idx_d[m]], sems[m], add=True)

            def swait(m):
                pltpu.make_async_copy(hr[m], acc_sh.at[idx_d[m]], sems[m]).wait()

            def compute(m):
                def edge(ei, _):
                    crow = cf[m][ei, :]
                    cb = crow.at[jnp.broadcast_to(hp, (16,))].get(
                        mode="promise_in_bounds")
                    for r in range(C // 16):
                        hr[m][ei, pl.ds(r * 16, 16)] = hr[m][ei, pl.ds(r * 16, 16)] * cb
                    return 0

                lax.fori_loop(0, BE, edge, 0, unroll=2)

            issue_idx(0, 0)
            issue_idx(1, 1)
            wait_idx(0, 0)
            gissue(hp, 0)

            def phase(kk, p):
                pn = (p + 2) % 3

                @pl.when(kk > 0)
                def _():
                    swait(pn)

                @pl.when(kk + 2 < nbatch)
                def _():
                    issue_idx(kk + 2, pn)

                p1 = (p + 1) % 3

                @pl.when(kk + 1 < nbatch)
                def _():
                    wait_idx(kk + 1, p1)
                    gissue(hp, p1)

                gwait(p)
                compute(p)
                sissue(p)

            def triple(tt, _):
                phase(3 * tt, 0)
                phase(3 * tt + 1, 1)
                phase(3 * tt + 2, 2)
                return 0

            lax.fori_loop(0, nbatch // 3, triple, 0)
            for kk in range(nbatch - nbatch % 3, nbatch):
                phase(kk, kk % 3)
            swait((nbatch - 1) % 3)
            plsc.subcore_barrier()
            pltpu.sync_copy(acc_sh.at[pl.ds(s * TROWS, TROWS)],
                            out_h.at[hp].at[pl.ds(s * TROWS, TROWS)])
            plsc.subcore_barrier()

    return k(h_byhead, src, dst, coef, z128)


def _gat_layer(h_byhead, asd, ads, gm, src, dst, z16, z128):
    ex, dpart = _edge_softmax(asd, ads, src, dst, gm[0], z16)
    coef = _coef_stage(ex, dst, dpart)
    return _edge_message(h_byhead, src, dst, coef, z128)


def kernel(x, edge_index, W1, att_src1, att_dst1, b1, W2, att_src2, att_dst2, b2):
    src = edge_index[0]
    dst = edge_index[1]
    z16 = jnp.zeros((TROWS, HW), jnp.float32)
    z128 = jnp.zeros((TROWS, C), jnp.float32)

    h1, asd1, ads1, gm1 = _dense_stage(x, W1, att_src1, att_dst1)
    out1 = _gat_layer(h1, asd1, ads1, gm1, src, dst, z16, z128)
    h2, asd2, ads2, gm2 = _mid_stage(out1, b1, W2, att_src2, att_dst2)
    out2 = _gat_layer(h2, asd2, ads2, gm2, src, dst, z16, z128)
    return _mean_stage(out2, b2)


# speedup vs baseline: 2.1164x; 1.0000x over previous
import functools

import jax
import jax.numpy as jnp
from jax import lax
from jax.experimental import pallas as pl
from jax.experimental.pallas import tpu as pltpu
from jax.experimental.pallas import tpu_sc as plsc

H = 8
HW = 16
C = 128
NCORES = 2
NSUB = 16
NP = 10240
TROWS = NP // NSUB
NB = 400
BE = 80


def _dense_core(h3, as_ref, ad_ref, asd_ref, ads_ref, gm_ref, i):
    asrc = jnp.sum(h3 * as_ref[...][None], axis=-1)
    adst = jnp.sum(h3 * ad_ref[...][None], axis=-1)
    asd_ref[...] = jnp.concatenate([asrc, adst], axis=-1)
    ads_ref[...] = jnp.concatenate([adst, asrc], axis=-1)
    am = jnp.concatenate([
        jnp.max(asrc, axis=0, keepdims=True),
        jnp.max(adst, axis=0, keepdims=True),
    ], axis=-1)
    am = jnp.broadcast_to(am, (8, HW))

    @pl.when(i == 0)
    def _():
        gm_ref[...] = jnp.full((8, HW), -1e30, jnp.float32)

    gm_ref[...] = jnp.maximum(gm_ref[...], am)


def _dense_body(x_ref, w_ref, as_ref, ad_ref, h_ref, asd_ref, ads_ref, gm_ref):
    i = pl.program_id(0)
    h = jnp.dot(x_ref[...], w_ref[...], preferred_element_type=jnp.float32)
    h3 = h.reshape(NB, H, C)
    h_ref[...] = h3.transpose(1, 0, 2)
    _dense_core(h3, as_ref, ad_ref, asd_ref, ads_ref, gm_ref, i)


def _dense_stage(x, w, att_src, att_dst):
    n = x.shape[0]
    k = x.shape[1]
    return pl.pallas_call(
        _dense_body,
        grid=(n // NB,),
        in_specs=[
            pl.BlockSpec((NB, k), lambda i: (i, 0)),
            pl.BlockSpec((k, H * C), lambda i: (0, 0)),
            pl.BlockSpec((H, C), lambda i: (0, 0)),
            pl.BlockSpec((H, C), lambda i: (0, 0)),
        ],
        out_specs=[
            pl.BlockSpec((H, NB, C), lambda i: (0, i, 0)),
            pl.BlockSpec((NB, HW), lambda i: (i, 0)),
            pl.BlockSpec((NB, HW), lambda i: (i, 0)),
            pl.BlockSpec((8, HW), lambda i: (0, 0)),
        ],
        out_shape=[
            jax.ShapeDtypeStruct((H, n, C), jnp.float32),
            jax.ShapeDtypeStruct((n, HW), jnp.float32),
            jax.ShapeDtypeStruct((n, HW), jnp.float32),
            jax.ShapeDtypeStruct((8, HW), jnp.float32),
        ],
    )(x, w, att_src, att_dst)


def _mid_body(o_ref, b_ref, w_ref, as_ref, ad_ref, h_ref, asd_ref, ads_ref, gm_ref):
    i = pl.program_id(0)
    m = jnp.sum(o_ref[...], axis=0) * (1.0 / H)
    z = jnp.maximum(m + b_ref[...], 0.0)
    h = jnp.dot(z, w_ref[...], preferred_element_type=jnp.float32)
    h3 = h.reshape(NB, H, C)
    h_ref[...] = h3.transpose(1, 0, 2)
    _dense_core(h3, as_ref, ad_ref, asd_ref, ads_ref, gm_ref, i)


def _mid_stage(out_byhead, b, w, att_src, att_dst):
    n = 10000
    return pl.pallas_call(
        _mid_body,
        grid=(n // NB,),
        in_specs=[
            pl.BlockSpec((H, NB, C), lambda i: (0, i, 0)),
            pl.BlockSpec((1, C), lambda i: (0, 0)),
            pl.BlockSpec((C, H * C), lambda i: (0, 0)),
            pl.BlockSpec((H, C), lambda i: (0, 0)),
            pl.BlockSpec((H, C), lambda i: (0, 0)),
        ],
        out_specs=[
            pl.BlockSpec((H, NB, C), lambda i: (0, i, 0)),
            pl.BlockSpec((NB, HW), lambda i: (i, 0)),
            pl.BlockSpec((NB, HW), lambda i: (i, 0)),
            pl.BlockSpec((8, HW), lambda i: (0, 0)),
        ],
        out_shape=[
            jax.ShapeDtypeStruct((H, n, C), jnp.float32),
            jax.ShapeDtypeStruct((n, HW), jnp.float32),
            jax.ShapeDtypeStruct((n, HW), jnp.float32),
            jax.ShapeDtypeStruct((8, HW), jnp.float32),
        ],
    )(out_byhead, b.reshape(1, C), w, att_src, att_dst)


def _mean_body(o_ref, b_ref, z_ref):
    z_ref[...] = jnp.sum(o_ref[...], axis=0) * (1.0 / H) + b_ref[...]


def _mean_stage(out_byhead, b):
    return pl.pallas_call(
        _mean_body,
        grid=(10000 // NB,),
        in_specs=[
            pl.BlockSpec((H, NB, C), lambda i: (0, i, 0)),
            pl.BlockSpec((1, C), lambda i: (0, 0)),
        ],
        out_specs=pl.BlockSpec((NB, C), lambda i: (i, 0)),
        out_shape=jax.ShapeDtypeStruct((10000, C), jnp.float32),
    )(out_byhead, b.reshape(1, C))


def _edge_softmax(asd, ads, src, dst, g16, z16):
    e = src.shape[0]
    ep = e // (NCORES * NSUB)
    nbatch = ep // BE
    mesh = plsc.VectorSubcoreMesh(core_axis_name="c", subcore_axis_name="s")

    @functools.partial(
        pl.kernel,
        out_type=[
            jax.ShapeDtypeStruct((e, HW), jnp.float32),
            jax.ShapeDtypeStruct((NCORES, NP, HW), jnp.float32),
        ],
        mesh=mesh,
        compiler_params=pltpu.CompilerParams(use_tc_tiling_on_sc=False),
        scratch_types=[
            [pltpu.VMEM((BE,), jnp.int32)] * 3,
            [pltpu.VMEM((BE,), jnp.int32)] * 3,
            [pltpu.VMEM((BE, HW), jnp.float32)] * 3,
            [pltpu.VMEM((BE, HW), jnp.float32)] * 3,
            [pltpu.VMEM((BE, HW), jnp.float32)] * 3,
            pltpu.VMEM((16,), jnp.float32),
            pltpu.VMEM_SHARED((NP, HW), jnp.float32),
            [pltpu.SemaphoreType.DMA] * 3,
            [pltpu.SemaphoreType.DMA] * 3,
            [pltpu.SemaphoreType.DMA] * 3,
            [pltpu.SemaphoreType.DMA] * 3,
        ],
    )
    def k(asd_h, ads_h, src_h, dst_h, g_h, z16_h, ex_h, dpart_h,
          idx_s, idx_d, as_v, ad_v, ex_v, g_v, den_sh, semi, sema, semw, sems):
        c = lax.axis_index("c")
        s = lax.axis_index("s")
        wid = c * NSUB + s
        pltpu.sync_copy(z16_h, den_sh.at[pl.ds(s * TROWS, TROWS)])
        pltpu.sync_copy(g_h, g_v)
        plsc.subcore_barrier()
        g = g_v[...]

        def ibase(kk):
            return pl.multiple_of(wid * ep + kk * BE, 8)

        def issue_idx(kk, m):
            pltpu.async_copy(src_h.at[pl.ds(ibase(kk), BE)], idx_s[m], semi[m])
            pltpu.async_copy(dst_h.at[pl.ds(ibase(kk), BE)], idx_d[m], semi[m])

        def wait_idx(kk, m):
            pltpu.make_async_copy(src_h.at[pl.ds(ibase(kk), BE)], idx_s[m], semi[m]).wait()
            pltpu.make_async_copy(dst_h.at[pl.ds(ibase(kk), BE)], idx_d[m], semi[m]).wait()

        def phase(kk, p):
            pn = (p + 2) % 3
            p1 = (p + 1) % 3

            @pl.when(kk > 0)
            def _():
                pltpu.make_async_copy(ex_v[pn], ex_h.at[pl.ds(ibase(kk - 1), BE)], semw[pn]).wait()
                pltpu.make_async_copy(ex_v[pn], den_sh.at[idx_d[pn]], sems[pn]).wait()

            @pl.when(kk + 2 < nbatch)
            def _():
                issue_idx(kk + 2, pn)

            @pl.when(kk + 1 < nbatch)
            def _():
                wait_idx(kk + 1, p1)
                pltpu.async_copy(asd_h.at[idx_s[p1]], as_v[p1], sema[p1])
                pltpu.async_copy(ads_h.at[idx_d[p1]], ad_v[p1], sema[p1])

            pltpu.make_async_copy(asd_h.at[idx_s[p]], as_v[p], sema[p]).wait()
            pltpu.make_async_copy(ads_h.at[idx_d[p]], ad_v[p], sema[p]).wait()

            def chunk(j, _):
                a_s = as_v[p][j, :]
                a_d = ad_v[p][j, :]
                tt = a_s + a_d
                alpha = jnp.where(tt >= 0, tt, 0.2 * tt)
                tb = g + a_d
                bound = jnp.where(tb >= 0, tb, 0.2 * tb)
                ex_v[p][j, :] = jnp.exp(alpha - bound)
                return 0

            lax.fori_loop(0, BE, chunk, 0, unroll=2)
            pltpu.async_copy(ex_v[p], ex_h.at[pl.ds(ibase(kk), BE)], semw[p])
            pltpu.async_copy(ex_v[p], den_sh.at[idx_d[p]], sems[p], add=True)

        issue_idx(0, 0)
        issue_idx(1, 1)
        wait_idx(0, 0)
        pltpu.async_copy(asd_h.at[idx_s[0]], as_v[0], sema[0])
        pltpu.async_copy(ads_h.at[idx_d[0]], ad_v[0], sema[0])

        def triple(tt, _):
            phase(3 * tt, 0)
            phase(3 * tt + 1, 1)
            phase(3 * tt + 2, 2)
            return 0

        lax.fori_loop(0, nbatch // 3, triple, 0)
        for kk in range(nbatch - nbatch % 3, nbatch):
            phase(kk, kk % 3)
        pm = (nbatch - 1) % 3
        pltpu.make_async_copy(ex_v[pm], ex_h.at[pl.ds(ibase(nbatch - 1), BE)], semw[pm]).wait()
        pltpu.make_async_copy(ex_v[pm], den_sh.at[idx_d[pm]], sems[pm]).wait()
        plsc.subcore_barrier()
        pltpu.sync_copy(den_sh.at[pl.ds(s * TROWS, TROWS)],
                        dpart_h.at[c].at[pl.ds(s * TROWS, TROWS)])

    return k(asd, ads, src, dst, g16, z16)


def _coef_stage(ex, dst, dpart):
    e = dst.shape[0]
    ep = e // (NCORES * NSUB)
    nbatch = ep // BE
    mesh = plsc.VectorSubcoreMesh(core_axis_name="c", subcore_axis_name="s")

    @functools.partial(
        pl.kernel,
        out_type=jax.ShapeDtypeStruct((e, HW), jnp.float32),
        mesh=mesh,
        compiler_params=pltpu.CompilerParams(use_tc_tiling_on_sc=False),
        scratch_types=[
            [pltpu.VMEM((BE,), jnp.int32)] * 3,
            [pltpu.VMEM((BE, HW), jnp.float32)] * 3,
            [pltpu.VMEM((BE, HW), jnp.float32)] * 3,
            [pltpu.VMEM((BE, HW), jnp.float32)] * 3,
            [pltpu.SemaphoreType.DMA] * 3,
            [pltpu.SemaphoreType.DMA] * 3,
            [pltpu.SemaphoreType.DMA] * 3,
        ],
    )
    def k(ex_h, dst_h, dpart_h, coef_h, idx_d, ex_v, e0, e1, semi, sema, semw):
        c = lax.axis_index("c")
        s = lax.axis_index("s")
        wid = c * NSUB + s

        def ibase(kk):
            return pl.multiple_of(wid * ep + kk * BE, 8)

        def phase(kk, p):
            pn = (p + 2) % 3
            p1 = (p + 1) % 3

            @pl.when(kk > 0)
            def _():
                pltpu.make_async_copy(ex_v[pn], coef_h.at[pl.ds(ibase(kk - 1), BE)], semw[pn]).wait()

            @pl.when(kk + 2 < nbatch)
            def _():
                pltpu.async_copy(dst_h.at[pl.ds(ibase(kk + 2), BE)], idx_d[pn], semi[pn])

            @pl.when(kk + 1 < nbatch)
            def _():
                pltpu.make_async_copy(dst_h.at[pl.ds(ibase(kk + 1), BE)], idx_d[p1], semi[p1]).wait()
                pltpu.async_copy(ex_h.at[pl.ds(ibase(kk + 1), BE)], ex_v[p1], sema[p1])
                pltpu.async_copy(dpart_h.at[0].at[idx_d[p1]], e0[p1], sema[p1])
                pltpu.async_copy(dpart_h.at[1].at[idx_d[p1]], e1[p1], sema[p1])

            pltpu.make_async_copy(ex_h.at[pl.ds(ibase(kk), BE)], ex_v[p], sema[p]).wait()
            pltpu.make_async_copy(dpart_h.at[0].at[idx_d[p]], e0[p], sema[p]).wait()
            pltpu.make_async_copy(dpart_h.at[1].at[idx_d[p]], e1[p], sema[p]).wait()

            def cdiv(j, _):
                ex_v[p][j, :] = ex_v[p][j, :] / (e0[p][j, :] + e1[p][j, :] + 1e-16)
                return 0

            lax.fori_loop(0, BE, cdiv, 0, unroll=2)
            pltpu.async_copy(ex_v[p], coef_h.at[pl.ds(ibase(kk), BE)], semw[p])

        pltpu.async_copy(dst_h.at[pl.ds(ibase(0), BE)], idx_d[0], semi[0])
        pltpu.async_copy(dst_h.at[pl.ds(ibase(1), BE)], idx_d[1], semi[1])
        pltpu.make_async_copy(dst_h.at[pl.ds(ibase(0), BE)], idx_d[0], semi[0]).wait()
        pltpu.async_copy(ex_h.at[pl.ds(ibase(0), BE)], ex_v[0], sema[0])
        pltpu.async_copy(dpart_h.at[0].at[idx_d[0]], e0[0], sema[0])
        pltpu.async_copy(dpart_h.at[1].at[idx_d[0]], e1[0], sema[0])

        def triple(tt, _):
            phase(3 * tt, 0)
            phase(3 * tt + 1, 1)
            phase(3 * tt + 2, 2)
            return 0

        lax.fori_loop(0, nbatch // 3, triple, 0)
        for kk in range(nbatch - nbatch % 3, nbatch):
            phase(kk, kk % 3)
        pm = (nbatch - 1) % 3
        pltpu.make_async_copy(ex_v[pm], coef_h.at[pl.ds(ibase(nbatch - 1), BE)], semw[pm]).wait()

    return k(ex, dst, dpart)


def _edge_message(h_byhead, src, dst, coef, z128):
    e = src.shape[0]
    ep = e // NSUB
    nbatch = ep // BE
    hperc = H // NCORES
    mesh = plsc.VectorSubcoreMesh(core_axis_name="c", subcore_axis_name="s")

    @functools.partial(
        pl.kernel,
        out_type=jax.ShapeDtypeStruct((H, NP, C), jnp.float32),
        mesh=mesh,
        compiler_params=pltpu.CompilerParams(use_tc_tiling_on_sc=False),
        scratch_types=[
            [pltpu.VMEM((BE,), jnp.int32)] * 3,
            [pltpu.VMEM((BE,), jnp.int32)] * 3,
            [pltpu.VMEM((BE, HW), jnp.float32)] * 3,
            [pltpu.VMEM((BE, C), jnp.float32)] * 3,
            pltpu.VMEM_SHARED((NP, C), jnp.float32),
            [pltpu.SemaphoreType.DMA] * 3,
            [pltpu.SemaphoreType.DMA] * 3,
            [pltpu.SemaphoreType.DMA] * 3,
        ],
    )
    def k(h_h, src_h, dst_h, coef_h, z128_h, out_h,
          idx_s, idx_d, cf, hr, acc_sh, semi, semh, sems):
        c = lax.axis_index("c")
        s = lax.axis_index("s")

        def issue_idx(kk, m):
            base = pl.multiple_of(s * ep + kk * BE, 8)
            pltpu.async_copy(src_h.at[pl.ds(base, BE)], idx_s[m], semi[m])
            pltpu.async_copy(dst_h.at[pl.ds(base, BE)], idx_d[m], semi[m])
            pltpu.async_copy(coef_h.at[pl.ds(base, BE)], cf[m], semi[m])

        def wait_idx(kk, m):
            base = pl.multiple_of(s * ep + kk * BE, 8)
            pltpu.make_async_copy(src_h.at[pl.ds(base, BE)], idx_s[m], semi[m]).wait()
            pltpu.make_async_copy(dst_h.at[pl.ds(base, BE)], idx_d[m], semi[m]).wait()
            pltpu.make_async_copy(coef_h.at[pl.ds(base, BE)], cf[m], semi[m]).wait()

        for hi in range(hperc):
            hp = c * hperc + hi
            pltpu.sync_copy(z128_h, acc_sh.at[pl.ds(s * TROWS, TROWS)])
            plsc.subcore_barrier()

            def gissue(hh, m):
                pltpu.async_copy(h_h.at[hp].at[idx_s[m]], hr[m], semh[m])

            def gwait(m):
                pltpu.make_async_copy(h_h.at[hp].at[idx_s[m]], hr[m], semh[m]).wait()

            def sissue(m):
                pltpu.async_copy(hr[m], acc_sh.at[idx_d[m]], sems[m], add=True)

            def swait(m):
                pltpu.make_async_copy(hr[m], acc_sh.at[idx_d[m]], sems[m]).wait()

            def compute(m):
                def edge(ei, _):
                    crow = cf[m][ei, :]
                    cb = crow.at[jnp.broadcast_to(hp, (16,))].get(
                        mode="promise_in_bounds")
                    for r in range(C // 16):
                        hr[m][ei, pl.ds(r * 16, 16)] = hr[m][ei, pl.ds(r * 16, 16)] * cb
                    return 0

                lax.fori_loop(0, BE, edge, 0, unroll=3)

            issue_idx(0, 0)
            issue_idx(1, 1)
            wait_idx(0, 0)
            gissue(hp, 0)

            def phase(kk, p):
                pn = (p + 2) % 3

                @pl.when(kk > 0)
                def _():
                    swait(pn)

                @pl.when(kk + 2 < nbatch)
                def _():
                    issue_idx(kk + 2, pn)

                p1 = (p + 1) % 3

                @pl.when(kk + 1 < nbatch)
                def _():
                    wait_idx(kk + 1, p1)
                    gissue(hp, p1)

                gwait(p)
                compute(p)
                sissue(p)

            def triple(tt, _):
                phase(3 * tt, 0)
                phase(3 * tt + 1, 1)
                phase(3 * tt + 2, 2)
                return 0

            lax.fori_loop(0, nbatch // 3, triple, 0)
            for kk in range(nbatch - nbatch % 3, nbatch):
                phase(kk, kk % 3)
            swait((nbatch - 1) % 3)
            plsc.subcore_barrier()
            pltpu.sync_copy(acc_sh.at[pl.ds(s * TROWS, TROWS)],
                            out_h.at[hp].at[pl.ds(s * TROWS, TROWS)])
            plsc.subcore_barrier()

    return k(h_byhead, src, dst, coef, z128)


def _gat_layer(h_byhead, asd, ads, gm, src, dst, z16, z128):
    ex, dpart = _edge_softmax(asd, ads, src, dst, gm[0], z16)
    coef = _coef_stage(ex, dst, dpart)
    return _edge_message(h_byhead, src, dst, coef, z128)


def kernel(x, edge_index, W1, att_src1, att_dst1, b1, W2, att_src2, att_dst2, b2):
    src = edge_index[0]
    dst = edge_index[1]
    z16 = jnp.zeros((TROWS, HW), jnp.float32)
    z128 = jnp.zeros((TROWS, C), jnp.float32)

    h1, asd1, ads1, gm1 = _dense_stage(x, W1, att_src1, att_dst1)
    out1 = _gat_layer(h1, asd1, ads1, gm1, src, dst, z16, z128)
    h2, asd2, ads2, gm2 = _mid_stage(out1, b1, W2, att_src2, att_dst2)
    out2 = _gat_layer(h2, asd2, ads2, gm2, src, dst, z16, z128)
    return _mean_stage(out2, b2)


# speedup vs baseline: 2.1281x; 1.0055x over previous
import functools

import jax
import jax.numpy as jnp
from jax import lax
from jax.experimental import pallas as pl
from jax.experimental.pallas import tpu as pltpu
from jax.experimental.pallas import tpu_sc as plsc

H = 8
HW = 16
C = 128
NCORES = 2
NSUB = 16
NP = 10240
TROWS = NP // NSUB
NB = 400
BE = 80


def _dense_core(h3, as_ref, ad_ref, asd_ref, ads_ref, gm_ref, i):
    asrc = jnp.sum(h3 * as_ref[...][None], axis=-1)
    adst = jnp.sum(h3 * ad_ref[...][None], axis=-1)
    asd_ref[...] = jnp.concatenate([asrc, adst], axis=-1)
    ads_ref[...] = jnp.concatenate([adst, asrc], axis=-1)
    am = jnp.concatenate([
        jnp.max(asrc, axis=0, keepdims=True),
        jnp.max(adst, axis=0, keepdims=True),
    ], axis=-1)
    am = jnp.broadcast_to(am, (8, HW))

    @pl.when(i == 0)
    def _():
        gm_ref[...] = jnp.full((8, HW), -1e30, jnp.float32)

    gm_ref[...] = jnp.maximum(gm_ref[...], am)


def _dense_body(x_ref, w_ref, as_ref, ad_ref, h_ref, asd_ref, ads_ref, gm_ref):
    i = pl.program_id(0)
    h = jnp.dot(x_ref[...], w_ref[...], preferred_element_type=jnp.float32)
    h3 = h.reshape(NB, H, C)
    h_ref[...] = h3.transpose(1, 0, 2)
    _dense_core(h3, as_ref, ad_ref, asd_ref, ads_ref, gm_ref, i)


def _dense_stage(x, w, att_src, att_dst):
    n = x.shape[0]
    k = x.shape[1]
    return pl.pallas_call(
        _dense_body,
        grid=(n // NB,),
        in_specs=[
            pl.BlockSpec((NB, k), lambda i: (i, 0)),
            pl.BlockSpec((k, H * C), lambda i: (0, 0)),
            pl.BlockSpec((H, C), lambda i: (0, 0)),
            pl.BlockSpec((H, C), lambda i: (0, 0)),
        ],
        out_specs=[
            pl.BlockSpec((H, NB, C), lambda i: (0, i, 0)),
            pl.BlockSpec((NB, HW), lambda i: (i, 0)),
            pl.BlockSpec((NB, HW), lambda i: (i, 0)),
            pl.BlockSpec((8, HW), lambda i: (0, 0)),
        ],
        out_shape=[
            jax.ShapeDtypeStruct((H, n, C), jnp.float32),
            jax.ShapeDtypeStruct((n, HW), jnp.float32),
            jax.ShapeDtypeStruct((n, HW), jnp.float32),
            jax.ShapeDtypeStruct((8, HW), jnp.float32),
        ],
    )(x, w, att_src, att_dst)


def _mid_body(o_ref, b_ref, w_ref, as_ref, ad_ref, h_ref, asd_ref, ads_ref, gm_ref):
    i = pl.program_id(0)
    m = jnp.sum(o_ref[...], axis=0) * (1.0 / H)
    z = jnp.maximum(m + b_ref[...], 0.0)
    h = jnp.dot(z, w_ref[...], preferred_element_type=jnp.float32)
    h3 = h.reshape(NB, H, C)
    h_ref[...] = h3.transpose(1, 0, 2)
    _dense_core(h3, as_ref, ad_ref, asd_ref, ads_ref, gm_ref, i)


def _mid_stage(out_byhead, b, w, att_src, att_dst):
    n = 10000
    return pl.pallas_call(
        _mid_body,
        grid=(n // NB,),
        in_specs=[
            pl.BlockSpec((H, NB, C), lambda i: (0, i, 0)),
            pl.BlockSpec((1, C), lambda i: (0, 0)),
            pl.BlockSpec((C, H * C), lambda i: (0, 0)),
            pl.BlockSpec((H, C), lambda i: (0, 0)),
            pl.BlockSpec((H, C), lambda i: (0, 0)),
        ],
        out_specs=[
            pl.BlockSpec((H, NB, C), lambda i: (0, i, 0)),
            pl.BlockSpec((NB, HW), lambda i: (i, 0)),
            pl.BlockSpec((NB, HW), lambda i: (i, 0)),
            pl.BlockSpec((8, HW), lambda i: (0, 0)),
        ],
        out_shape=[
            jax.ShapeDtypeStruct((H, n, C), jnp.float32),
            jax.ShapeDtypeStruct((n, HW), jnp.float32),
            jax.ShapeDtypeStruct((n, HW), jnp.float32),
            jax.ShapeDtypeStruct((8, HW), jnp.float32),
        ],
    )(out_byhead, b.reshape(1, C), w, att_src, att_dst)


def _mean_body(o_ref, b_ref, z_ref):
    z_ref[...] = jnp.sum(o_ref[...], axis=0) * (1.0 / H) + b_ref[...]


def _mean_stage(out_byhead, b):
    return pl.pallas_call(
        _mean_body,
        grid=(10000 // NB,),
        in_specs=[
            pl.BlockSpec((H, NB, C), lambda i: (0, i, 0)),
            pl.BlockSpec((1, C), lambda i: (0, 0)),
        ],
        out_specs=pl.BlockSpec((NB, C), lambda i: (i, 0)),
        out_shape=jax.ShapeDtypeStruct((10000, C), jnp.float32),
    )(out_byhead, b.reshape(1, C))


def _edge_softmax(asd, ads, src, dst, g16, z16):
    e = src.shape[0]
    ep = e // (NCORES * NSUB)
    nbatch = ep // BE
    mesh = plsc.VectorSubcoreMesh(core_axis_name="c", subcore_axis_name="s")

    @functools.partial(
        pl.kernel,
        out_type=[
            jax.ShapeDtypeStruct((e, HW), jnp.float32),
            jax.ShapeDtypeStruct((NCORES, NP, HW), jnp.float32),
        ],
        mesh=mesh,
        compiler_params=pltpu.CompilerParams(use_tc_tiling_on_sc=False),
        scratch_types=[
            [pltpu.VMEM((BE,), jnp.int32)] * 3,
            [pltpu.VMEM((BE,), jnp.int32)] * 3,
            [pltpu.VMEM((BE, HW), jnp.float32)] * 3,
            [pltpu.VMEM((BE, HW), jnp.float32)] * 3,
            [pltpu.VMEM((BE, HW), jnp.float32)] * 3,
            pltpu.VMEM((16,), jnp.float32),
            pltpu.VMEM_SHARED((NP, HW), jnp.float32),
            [pltpu.SemaphoreType.DMA] * 3,
            [pltpu.SemaphoreType.DMA] * 3,
            [pltpu.SemaphoreType.DMA] * 3,
            [pltpu.SemaphoreType.DMA] * 3,
        ],
    )
    def k(asd_h, ads_h, src_h, dst_h, g_h, z16_h, ex_h, dpart_h,
          idx_s, idx_d, as_v, ad_v, ex_v, g_v, den_sh, semi, sema, semw, sems):
        c = lax.axis_index("c")
        s = lax.axis_index("s")
        wid = c * NSUB + s
        pltpu.sync_copy(z16_h, den_sh.at[pl.ds(s * TROWS, TROWS)])
        pltpu.sync_copy(g_h, g_v)
        plsc.subcore_barrier()
        g = g_v[...]

        def ibase(kk):
            return pl.multiple_of(wid * ep + kk * BE, 8)

        def issue_idx(kk, m):
            pltpu.async_copy(src_h.at[pl.ds(ibase(kk), BE)], idx_s[m], semi[m])
            pltpu.async_copy(dst_h.at[pl.ds(ibase(kk), BE)], idx_d[m], semi[m])

        def wait_idx(kk, m):
            pltpu.make_async_copy(src_h.at[pl.ds(ibase(kk), BE)], idx_s[m], semi[m]).wait()
            pltpu.make_async_copy(dst_h.at[pl.ds(ibase(kk), BE)], idx_d[m], semi[m]).wait()

        def phase(kk, p):
            pn = (p + 2) % 3
            p1 = (p + 1) % 3

            @pl.when(kk > 0)
            def _():
                pltpu.make_async_copy(ex_v[pn], ex_h.at[pl.ds(ibase(kk - 1), BE)], semw[pn]).wait()
                pltpu.make_async_copy(ex_v[pn], den_sh.at[idx_d[pn]], sems[pn]).wait()

            @pl.when(kk + 2 < nbatch)
            def _():
                issue_idx(kk + 2, pn)

            @pl.when(kk + 1 < nbatch)
            def _():
                wait_idx(kk + 1, p1)
                pltpu.async_copy(asd_h.at[idx_s[p1]], as_v[p1], sema[p1])
                pltpu.async_copy(ads_h.at[idx_d[p1]], ad_v[p1], sema[p1])

            pltpu.make_async_copy(asd_h.at[idx_s[p]], as_v[p], sema[p]).wait()
            pltpu.make_async_copy(ads_h.at[idx_d[p]], ad_v[p], sema[p]).wait()

            def chunk(j, _):
                a_s = as_v[p][j, :]
                a_d = ad_v[p][j, :]
                tt = a_s + a_d
                alpha = jnp.where(tt >= 0, tt, 0.2 * tt)
                tb = g + a_d
                bound = jnp.where(tb >= 0, tb, 0.2 * tb)
                ex_v[p][j, :] = jnp.exp(alpha - bound)
                return 0

            lax.fori_loop(0, BE, chunk, 0, unroll=2)
            pltpu.async_copy(ex_v[p], ex_h.at[pl.ds(ibase(kk), BE)], semw[p])
            pltpu.async_copy(ex_v[p], den_sh.at[idx_d[p]], sems[p], add=True)

        issue_idx(0, 0)
        issue_idx(1, 1)
        wait_idx(0, 0)
        pltpu.async_copy(asd_h.at[idx_s[0]], as_v[0], sema[0])
        pltpu.async_copy(ads_h.at[idx_d[0]], ad_v[0], sema[0])

        def triple(tt, _):
            phase(3 * tt, 0)
            phase(3 * tt + 1, 1)
            phase(3 * tt + 2, 2)
            return 0

        lax.fori_loop(0, nbatch // 3, triple, 0)
        for kk in range(nbatch - nbatch % 3, nbatch):
            phase(kk, kk % 3)
        pm = (nbatch - 1) % 3
        pltpu.make_async_copy(ex_v[pm], ex_h.at[pl.ds(ibase(nbatch - 1), BE)], semw[pm]).wait()
        pltpu.make_async_copy(ex_v[pm], den_sh.at[idx_d[pm]], sems[pm]).wait()
        plsc.subcore_barrier()
        pltpu.sync_copy(den_sh.at[pl.ds(s * TROWS, TROWS)],
                        dpart_h.at[c].at[pl.ds(s * TROWS, TROWS)])

    return k(asd, ads, src, dst, g16, z16)


def _coef_stage(ex, dst, dpart):
    e = dst.shape[0]
    ep = e // (NCORES * NSUB)
    nbatch = ep // BE
    mesh = plsc.VectorSubcoreMesh(core_axis_name="c", subcore_axis_name="s")

    @functools.partial(
        pl.kernel,
        out_type=jax.ShapeDtypeStruct((e, HW), jnp.float32),
        mesh=mesh,
        compiler_params=pltpu.CompilerParams(use_tc_tiling_on_sc=False),
        scratch_types=[
            [pltpu.VMEM((BE,), jnp.int32)] * 3,
            [pltpu.VMEM((BE, HW), jnp.float32)] * 3,
            [pltpu.VMEM((BE, HW), jnp.float32)] * 3,
            [pltpu.VMEM((BE, HW), jnp.float32)] * 3,
            [pltpu.SemaphoreType.DMA] * 3,
            [pltpu.SemaphoreType.DMA] * 3,
            [pltpu.SemaphoreType.DMA] * 3,
        ],
    )
    def k(ex_h, dst_h, dpart_h, coef_h, idx_d, ex_v, e0, e1, semi, sema, semw):
        c = lax.axis_index("c")
        s = lax.axis_index("s")
        wid = c * NSUB + s

        def ibase(kk):
            return pl.multiple_of(wid * ep + kk * BE, 8)

        def phase(kk, p):
            pn = (p + 2) % 3
            p1 = (p + 1) % 3

            @pl.when(kk > 0)
            def _():
                pltpu.make_async_copy(ex_v[pn], coef_h.at[pl.ds(ibase(kk - 1), BE)], semw[pn]).wait()

            @pl.when(kk + 2 < nbatch)
            def _():
                pltpu.async_copy(dst_h.at[pl.ds(ibase(kk + 2), BE)], idx_d[pn], semi[pn])

            @pl.when(kk + 1 < nbatch)
            def _():
                pltpu.make_async_copy(dst_h.at[pl.ds(ibase(kk + 1), BE)], idx_d[p1], semi[p1]).wait()
                pltpu.async_copy(ex_h.at[pl.ds(ibase(kk + 1), BE)], ex_v[p1], sema[p1])
                pltpu.async_copy(dpart_h.at[0].at[idx_d[p1]], e0[p1], sema[p1])
                pltpu.async_copy(dpart_h.at[1].at[idx_d[p1]], e1[p1], sema[p1])

            pltpu.make_async_copy(ex_h.at[pl.ds(ibase(kk), BE)], ex_v[p], sema[p]).wait()
            pltpu.make_async_copy(dpart_h.at[0].at[idx_d[p]], e0[p], sema[p]).wait()
            pltpu.make_async_copy(dpart_h.at[1].at[idx_d[p]], e1[p], sema[p]).wait()

            def cdiv(j, _):
                ex_v[p][j, :] = ex_v[p][j, :] / (e0[p][j, :] + e1[p][j, :] + 1e-16)
                return 0

            lax.fori_loop(0, BE, cdiv, 0, unroll=2)
            pltpu.async_copy(ex_v[p], coef_h.at[pl.ds(ibase(kk), BE)], semw[p])

        pltpu.async_copy(dst_h.at[pl.ds(ibase(0), BE)], idx_d[0], semi[0])
        pltpu.async_copy(dst_h.at[pl.ds(ibase(1), BE)], idx_d[1], semi[1])
        pltpu.make_async_copy(dst_h.at[pl.ds(ibase(0), BE)], idx_d[0], semi[0]).wait()
        pltpu.async_copy(ex_h.at[pl.ds(ibase(0), BE)], ex_v[0], sema[0])
        pltpu.async_copy(dpart_h.at[0].at[idx_d[0]], e0[0], sema[0])
        pltpu.async_copy(dpart_h.at[1].at[idx_d[0]], e1[0], sema[0])

        def triple(tt, _):
            phase(3 * tt, 0)
            phase(3 * tt + 1, 1)
            phase(3 * tt + 2, 2)
            return 0

        lax.fori_loop(0, nbatch // 3, triple, 0)
        for kk in range(nbatch - nbatch % 3, nbatch):
            phase(kk, kk % 3)
        pm = (nbatch - 1) % 3
        pltpu.make_async_copy(ex_v[pm], coef_h.at[pl.ds(ibase(nbatch - 1), BE)], semw[pm]).wait()

    return k(ex, dst, dpart)


def _edge_message(h_byhead, src, dst, coef, z128):
    e = src.shape[0]
    ep = e // NSUB
    nbatch = ep // BE
    hperc = H // NCORES
    mesh = plsc.VectorSubcoreMesh(core_axis_name="c", subcore_axis_name="s")

    @functools.partial(
        pl.kernel,
        out_type=jax.ShapeDtypeStruct((H, NP, C), jnp.float32),
        mesh=mesh,
        compiler_params=pltpu.CompilerParams(use_tc_tiling_on_sc=False),
        scratch_types=[
            [pltpu.VMEM((BE,), jnp.int32)] * 3,
            [pltpu.VMEM((BE,), jnp.int32)] * 3,
            [pltpu.VMEM((BE, HW), jnp.float32)] * 3,
            [pltpu.VMEM((BE, C), jnp.float32)] * 3,
            pltpu.VMEM_SHARED((NP, C), jnp.float32),
            [pltpu.SemaphoreType.DMA] * 3,
            [pltpu.SemaphoreType.DMA] * 3,
            [pltpu.SemaphoreType.DMA] * 3,
        ],
    )
    def k(h_h, src_h, dst_h, coef_h, z128_h, out_h,
          idx_s, idx_d, cf, hr, acc_sh, semi, semh, sems):
        c = lax.axis_index("c")
        s = lax.axis_index("s")

        def issue_idx(kk, m):
            base = pl.multiple_of(s * ep + kk * BE, 8)
            pltpu.async_copy(src_h.at[pl.ds(base, BE)], idx_s[m], semi[m])
            pltpu.async_copy(dst_h.at[pl.ds(base, BE)], idx_d[m], semi[m])
            pltpu.async_copy(coef_h.at[pl.ds(base, BE)], cf[m], semi[m])

        def wait_idx(kk, m):
            base = pl.multiple_of(s * ep + kk * BE, 8)
            pltpu.make_async_copy(src_h.at[pl.ds(base, BE)], idx_s[m], semi[m]).wait()
            pltpu.make_async_copy(dst_h.at[pl.ds(base, BE)], idx_d[m], semi[m]).wait()
            pltpu.make_async_copy(coef_h.at[pl.ds(base, BE)], cf[m], semi[m]).wait()

        for hi in range(hperc):
            hp = c * hperc + hi
            pltpu.sync_copy(z128_h, acc_sh.at[pl.ds(s * TROWS, TROWS)])
            plsc.subcore_barrier()

            def gissue(hh, m):
                pltpu.async_copy(h_h.at[hp].at[idx_s[m]], hr[m], semh[m])

            def gwait(m):
                pltpu.make_async_copy(h_h.at[hp].at[idx_s[m]], hr[m], semh[m]).wait()

            def sissue(m):
                pltpu.async_copy(hr[m], acc_sh.at[idx_d[m]], sems[m], add=True)

            def swait(m):
                pltpu.make_async_copy(hr[m], acc_sh.at[idx_d[m]], sems[m]).wait()

            def compute(m):
                def edge(ei, _):
                    crow = cf[m][ei, :]
                    cb = crow.at[jnp.broadcast_to(hp, (16,))].get(
                        mode="promise_in_bounds")
                    for r in range(C // 16):
                        hr[m][ei, pl.ds(r * 16, 16)] = hr[m][ei, pl.ds(r * 16, 16)] * cb
                    return 0

                lax.fori_loop(0, BE, edge, 0, unroll=2)

            issue_idx(0, 0)
            issue_idx(1, 1)
            wait_idx(0, 0)
            gissue(hp, 0)

            def phase(kk, p):
                pn = (p + 2) % 3

                @pl.when(kk > 0)
                def _():
                    swait(pn)

                @pl.when(kk + 2 < nbatch)
                def _():
                    issue_idx(kk + 2, pn)

                p1 = (p + 1) % 3

                @pl.when(kk + 1 < nbatch)
                def _():
                    wait_idx(kk + 1, p1)
                    gissue(hp, p1)

                gwait(p)
                compute(p)
                sissue(p)

            def triple(tt, _):
                phase(3 * tt, 0)
                phase(3 * tt + 1, 1)
                phase(3 * tt + 2, 2)
                return 0

            lax.fori_loop(0, nbatch // 3, triple, 0)
            for kk in range(nbatch - nbatch % 3, nbatch):
                phase(kk, kk % 3)
            swait((nbatch - 1) % 3)
            plsc.subcore_barrier()
            pltpu.sync_copy(acc_sh.at[pl.ds(s * TROWS, TROWS)],
                            out_h.at[hp].at[pl.ds(s * TROWS, TROWS)])
            plsc.subcore_barrier()

    return k(h_byhead, src, dst, coef, z128)


def _gat_layer(h_byhead, asd, ads, gm, src, dst, z16, z128):
    ex, dpart = _edge_softmax(asd, ads, src, dst, gm[0], z16)
    coef = _coef_stage(ex, dst, dpart)
    return _edge_message(h_byhead, src, dst, coef, z128)


def kernel(x, edge_index, W1, att_src1, att_dst1, b1, W2, att_src2, att_dst2, b2):
    src = edge_index[0]
    dst = edge_index[1]
    z16 = jnp.zeros((TROWS, HW), jnp.float32)
    z128 = jnp.zeros((TROWS, C), jnp.float32)

    h1, asd1, ads1, gm1 = _dense_stage(x, W1, att_src1, att_dst1)
    out1 = _gat_layer(h1, asd1, ads1, gm1, src, dst, z16, z128)
    h2, asd2, ads2, gm2 = _mid_stage(out1, b1, W2, att_src2, att_dst2)
    out2 = _gat_layer(h2, asd2, ads2, gm2, src, dst, z16, z128)
    return _mean_stage(out2, b2)
